# baseline (speedup 1.0000x reference)
import functools
import jax
import jax.numpy as jnp
import numpy as np
from jax import lax
from jax.experimental import pallas as pl
from jax.experimental.pallas import tpu as pltpu

N = 50000
E = 1600000
N_PAD = 50176
R = 1792
NBLK = N_PAD // R


def _lrelu(v, a):
    return jnp.where(v >= 0, v, a * v)


def _scales(dego, degi):
    dout = lax.rsqrt(jnp.maximum(dego[0] + dego[1], 1.0))
    din = lax.rsqrt(jnp.maximum(degi[0] + degi[1], 1.0))
    return dout, din


def _t1_body(dego_ref, degi_ref, x_ref, w1_ref, o_ref):
    dout, _ = _scales(dego_ref[...], degi_ref[...])
    o_ref[...] = jnp.dot(x_ref[...] * dout[:, None], w1_ref[...],
                         preferred_element_type=jnp.float32)


def _mid_body(dego_ref, degi_ref, agg_ref, w_ref, b_ref, o_ref, *, has_w):
    dout, din = _scales(dego_ref[...], degi_ref[...])
    h = (agg_ref[0] + agg_ref[1]) * din[:, None]
    if has_w:
        h = jnp.dot(h, w_ref[...], preferred_element_type=jnp.float32)
    h = _lrelu(h + b_ref[...], 0.01)
    o_ref[...] = h * dout[:, None]


def _split_body(dego_ref, degi_ref, agg_ref, w_ref, b_ref, oa_ref, ob_ref):
    dout, din = _scales(dego_ref[...], degi_ref[...])
    h = (agg_ref[0] + agg_ref[1]) * din[:, None]
    h = jnp.dot(h, w_ref[...], preferred_element_type=jnp.float32)
    h = _lrelu(h + b_ref[...], 0.01) * dout[:, None]
    oa_ref[...] = h[:, :32]
    ob_ref[...] = h[:, 32:]


def _head_body(dego_ref, degi_ref, aa_ref, ab_ref, w4_ref, b4_ref, gb_ref,
               f1w_ref, f1b_ref, f2w_ref, f2b_ref, f3w_ref, f3b_ref,
               o_ref, acc_ref):
    i = pl.program_id(0)
    _, din = _scales(dego_ref[...], degi_ref[...])
    h = (jnp.dot(aa_ref[0] + aa_ref[1], w4_ref[:32, :],
                 preferred_element_type=jnp.float32)
         + jnp.dot(ab_ref[0] + ab_ref[1], w4_ref[32:, :],
                   preferred_element_type=jnp.float32))
    h = h * din[:, None] + b4_ref[...]
    h = h * gb_ref[0:1, :] + gb_ref[1:2, :]
    z = _lrelu(h, 0.01)
    rows = i * R + lax.broadcasted_iota(jnp.int32, (R, 1), 0)
    z = jnp.where(rows < N, z, 0.0)
    part = jnp.sum(z, axis=0, keepdims=True)

    @pl.when(i == 0)
    def _():
        acc_ref[...] = jnp.zeros_like(acc_ref)

    acc_ref[...] += part

    @pl.when(i == NBLK - 1)
    def _():
        g = acc_ref[...]
        o = _lrelu(jnp.dot(g, f1w_ref[...]) + f1b_ref[...], 0.1)
        o = _lrelu(jnp.dot(o, f2w_ref[...]) + f2b_ref[...], 0.1)
        o_ref[...] = jnp.dot(o, f3w_ref[...]) + f3b_ref[...]


def _deg_spec():
    return pl.BlockSpec((2, R), lambda i: (0, i))


def _full(shape):
    return pl.BlockSpec(shape, lambda i: tuple(0 for _ in shape))


def _tc_t1(dego, degi, x_pad, w1):
    return pl.pallas_call(
        _t1_body,
        grid=(NBLK,),
        in_specs=[_deg_spec(), _deg_spec(),
                  pl.BlockSpec((R, 8), lambda i: (i, 0)),
                  _full((8, 16))],
        out_specs=pl.BlockSpec((R, 16), lambda i: (i, 0)),
        out_shape=jax.ShapeDtypeStruct((N_PAD, 16), jnp.float32),
    )(dego, degi, x_pad, w1)


def _tc_mid(dego, degi, agg, w, b, w_in, w_out):
    in_specs = [_deg_spec(), _deg_spec(),
                pl.BlockSpec((2, R, w_in), lambda i: (0, i, 0))]
    args = [dego, degi, agg]
    if w is not None:
        in_specs.append(_full((w_in, w_out)))
        args.append(w)
        body = functools.partial(_mid_body, has_w=True)
    else:
        body = (lambda dego_ref, degi_ref, agg_ref, b_ref, o_ref:
                _mid_body(dego_ref, degi_ref, agg_ref, None, b_ref, o_ref,
                          has_w=False))
    in_specs.append(_full((1, w_out)))
    args.append(b.reshape(1, -1))
    return pl.pallas_call(
        body,
        grid=(NBLK,),
        in_specs=in_specs,
        out_specs=pl.BlockSpec((R, w_out), lambda i: (i, 0)),
        out_shape=jax.ShapeDtypeStruct((N_PAD, w_out), jnp.float32),
    )(*args)


def _tc_split(dego, degi, agg, w3, b3):
    return pl.pallas_call(
        _split_body,
        grid=(NBLK,),
        in_specs=[_deg_spec(), _deg_spec(),
                  pl.BlockSpec((2, R, 32), lambda i: (0, i, 0)),
                  _full((32, 64)), _full((1, 64))],
        out_specs=[pl.BlockSpec((R, 32), lambda i: (i, 0)),
                   pl.BlockSpec((R, 32), lambda i: (i, 0))],
        out_shape=[jax.ShapeDtypeStruct((N_PAD, 32), jnp.float32),
                   jax.ShapeDtypeStruct((N_PAD, 32), jnp.float32)],
    )(dego, degi, agg, w3, b3.reshape(1, -1))


def _tc_head(dego, degi, agga, aggb, w4, b4, gamma, beta,
             f1w, f1b, f2w, f2b, f3w, f3b):
    gb = jnp.stack([gamma * np.float32(1.0 / np.sqrt(1.0 + 1e-5)), beta])
    return pl.pallas_call(
        _head_body,
        grid=(NBLK,),
        in_specs=[_deg_spec(), _deg_spec(),
                  pl.BlockSpec((2, R, 32), lambda i: (0, i, 0)),
                  pl.BlockSpec((2, R, 32), lambda i: (0, i, 0)),
                  _full((64, 50)), _full((1, 50)), _full((2, 50)),
                  _full((50, 30)), _full((1, 30)),
                  _full((30, 10)), _full((1, 10)),
                  _full((10, 2)), _full((1, 2))],
        out_specs=pl.BlockSpec((1, 2), lambda i: (0, 0)),
        out_shape=jax.ShapeDtypeStruct((1, 2), jnp.float32),
        scratch_shapes=[pltpu.VMEM((1, 50), jnp.float32)],
        compiler_params=pltpu.CompilerParams(
            dimension_semantics=("arbitrary",)),
    )(dego, degi, agga, aggb, w4, b4.reshape(1, -1), gb,
      f1w, f1b.reshape(1, -1), f2w, f2b.reshape(1, -1),
      f3w, f3b.reshape(1, -1))


def _degrees(src, dst):
    dego = jnp.zeros((2, N_PAD), jnp.float32).at[0, src].add(1.0)
    degi = jnp.zeros((2, N_PAD), jnp.float32).at[0, dst].add(1.0)
    return dego, degi


def _mp(t, src, dst, ew, w):
    m = t[src] * ew[:, None]
    agg = jnp.zeros((2, N_PAD, w), jnp.float32).at[0, dst].add(m)
    return agg


def kernel(x, edge_index, edge_weights, W1, b1, W2, b2, W3, b3, W4, b4,
           gamma, beta, fc1_W, fc1_b, fc2_W, fc2_b, fc3_W, fc3_b):
    src = edge_index[0].astype(jnp.int32)
    dst = edge_index[1].astype(jnp.int32)
    ew = edge_weights

    x_pad = jnp.zeros((N_PAD, 8), jnp.float32).at[:N, :6].set(x)
    w1_pad = jnp.zeros((8, 16), jnp.float32).at[:6].set(W1)

    dego, degi = _degrees(src, dst)

    t1 = _tc_t1(dego, degi, x_pad, w1_pad)
    agg1 = _mp(t1, src, dst, ew, 16)
    t2 = _tc_mid(dego, degi, agg1, None, b1, 16, 16)
    agg2 = _mp(t2, src, dst, ew, 16)
    t3 = _tc_mid(dego, degi, agg2, W2, b2, 16, 32)
    agg3 = _mp(t3, src, dst, ew, 32)
    t4a, t4b = _tc_split(dego, degi, agg3, W3, b3)
    agg4a = _mp(t4a, src, dst, ew, 32)
    agg4b = _mp(t4b, src, dst, ew, 32)
    return _tc_head(dego, degi, agg4a, agg4b, W4, b4, gamma, beta,
                    fc1_W, fc1_b, fc2_W, fc2_b, fc3_W, fc3_b)


# speedup vs baseline: 9.1211x; 9.1211x over previous
import functools
import jax
import jax.numpy as jnp
import numpy as np
from jax import lax
from jax.experimental import pallas as pl
from jax.experimental.pallas import tpu as pltpu
from jax.experimental.pallas import tpu_sc as plsc

N = 50000
E = 1600000
N_PAD = 50176
R = 1792
NBLK = N_PAD // R

NC = 2
NS = 16
NW = NC * NS
SZ = 128
RW = 400
E_PAD = NW * RW * SZ
K = 16
NCH = RW // K
NT = N_PAD // NS


def _lrelu(v, a):
    return jnp.where(v >= 0, v, a * v)


def _scales(dego, degi):
    dout = lax.rsqrt(jnp.maximum(dego[0] + dego[1], 1.0))
    din = lax.rsqrt(jnp.maximum(degi[0] + degi[1], 1.0))
    return dout, din


def _t1_body(dego_ref, degi_ref, x_ref, w1_ref, o_ref):
    dout, _ = _scales(dego_ref[...], degi_ref[...])
    o_ref[...] = jnp.dot(x_ref[...] * dout[:, None], w1_ref[...],
                         preferred_element_type=jnp.float32)


def _mid_body(dego_ref, degi_ref, agg_ref, w_ref, b_ref, o_ref, *, has_w):
    dout, din = _scales(dego_ref[...], degi_ref[...])
    h = (agg_ref[0] + agg_ref[1]) * din[:, None]
    if has_w:
        h = jnp.dot(h, w_ref[...], preferred_element_type=jnp.float32)
    h = _lrelu(h + b_ref[...], 0.01)
    o_ref[...] = h * dout[:, None]


def _split_body(dego_ref, degi_ref, agg_ref, w_ref, b_ref, oa_ref, ob_ref):
    dout, din = _scales(dego_ref[...], degi_ref[...])
    h = (agg_ref[0] + agg_ref[1]) * din[:, None]
    h = jnp.dot(h, w_ref[...], preferred_element_type=jnp.float32)
    h = _lrelu(h + b_ref[...], 0.01) * dout[:, None]
    oa_ref[...] = h[:, :32]
    ob_ref[...] = h[:, 32:]


def _head_body(dego_ref, degi_ref, aa_ref, ab_ref, w4_ref, b4_ref, gb_ref,
               f1w_ref, f1b_ref, f2w_ref, f2b_ref, f3w_ref, f3b_ref,
               o_ref, acc_ref):
    i = pl.program_id(0)
    _, din = _scales(dego_ref[...], degi_ref[...])
    h = (jnp.dot(aa_ref[0] + aa_ref[1], w4_ref[:32, :],
                 preferred_element_type=jnp.float32)
         + jnp.dot(ab_ref[0] + ab_ref[1], w4_ref[32:, :],
                   preferred_element_type=jnp.float32))
    h = h * din[:, None] + b4_ref[...]
    h = h * gb_ref[0:1, :] + gb_ref[1:2, :]
    z = _lrelu(h, 0.01)
    rows = i * R + lax.broadcasted_iota(jnp.int32, (R, 1), 0)
    z = jnp.where(rows < N, z, 0.0)
    part = jnp.sum(z, axis=0, keepdims=True)

    @pl.when(i == 0)
    def _():
        acc_ref[...] = jnp.zeros_like(acc_ref)

    acc_ref[...] += part

    @pl.when(i == NBLK - 1)
    def _():
        g = acc_ref[...]
        o = _lrelu(jnp.dot(g, f1w_ref[...]) + f1b_ref[...], 0.1)
        o = _lrelu(jnp.dot(o, f2w_ref[...]) + f2b_ref[...], 0.1)
        o_ref[...] = jnp.dot(o, f3w_ref[...]) + f3b_ref[...]


def _deg_spec():
    return pl.BlockSpec((2, R), lambda i: (0, i))


def _full(shape):
    return pl.BlockSpec(shape, lambda i: tuple(0 for _ in shape))


def _tc_t1(dego, degi, x_pad, w1):
    return pl.pallas_call(
        _t1_body,
        grid=(NBLK,),
        in_specs=[_deg_spec(), _deg_spec(),
                  pl.BlockSpec((R, 8), lambda i: (i, 0)),
                  _full((8, 16))],
        out_specs=pl.BlockSpec((R, 16), lambda i: (i, 0)),
        out_shape=jax.ShapeDtypeStruct((N_PAD, 16), jnp.float32),
    )(dego, degi, x_pad, w1)


def _tc_mid(dego, degi, agg, w, b, w_in, w_out):
    in_specs = [_deg_spec(), _deg_spec(),
                pl.BlockSpec((2, R, w_in), lambda i: (0, i, 0))]
    args = [dego, degi, agg]
    if w is not None:
        in_specs.append(_full((w_in, w_out)))
        args.append(w)
        body = functools.partial(_mid_body, has_w=True)
    else:
        body = (lambda dego_ref, degi_ref, agg_ref, b_ref, o_ref:
                _mid_body(dego_ref, degi_ref, agg_ref, None, b_ref, o_ref,
                          has_w=False))
    in_specs.append(_full((1, w_out)))
    args.append(b.reshape(1, -1))
    return pl.pallas_call(
        body,
        grid=(NBLK,),
        in_specs=in_specs,
        out_specs=pl.BlockSpec((R, w_out), lambda i: (i, 0)),
        out_shape=jax.ShapeDtypeStruct((N_PAD, w_out), jnp.float32),
    )(*args)


def _tc_split(dego, degi, agg, w3, b3):
    return pl.pallas_call(
        _split_body,
        grid=(NBLK,),
        in_specs=[_deg_spec(), _deg_spec(),
                  pl.BlockSpec((2, R, 32), lambda i: (0, i, 0)),
                  _full((32, 64)), _full((1, 64))],
        out_specs=[pl.BlockSpec((R, 32), lambda i: (i, 0)),
                   pl.BlockSpec((R, 32), lambda i: (i, 0))],
        out_shape=[jax.ShapeDtypeStruct((N_PAD, 32), jnp.float32),
                   jax.ShapeDtypeStruct((N_PAD, 32), jnp.float32)],
    )(dego, degi, agg, w3, b3.reshape(1, -1))


def _tc_head(dego, degi, agga, aggb, w4, b4, gamma, beta,
             f1w, f1b, f2w, f2b, f3w, f3b):
    gb = jnp.stack([gamma * np.float32(1.0 / np.sqrt(1.0 + 1e-5)), beta])
    return pl.pallas_call(
        _head_body,
        grid=(NBLK,),
        in_specs=[_deg_spec(), _deg_spec(),
                  pl.BlockSpec((2, R, 32), lambda i: (0, i, 0)),
                  pl.BlockSpec((2, R, 32), lambda i: (0, i, 0)),
                  _full((64, 50)), _full((1, 50)), _full((2, 50)),
                  _full((50, 30)), _full((1, 30)),
                  _full((30, 10)), _full((1, 10)),
                  _full((10, 2)), _full((1, 2))],
        out_specs=pl.BlockSpec((1, 2), lambda i: (0, 0)),
        out_shape=jax.ShapeDtypeStruct((1, 2), jnp.float32),
        scratch_shapes=[pltpu.VMEM((1, 50), jnp.float32)],
        compiler_params=pltpu.CompilerParams(
            dimension_semantics=("arbitrary",)),
    )(dego, degi, agga, aggb, w4, b4.reshape(1, -1), gb,
      f1w, f1b.reshape(1, -1), f2w, f2b.reshape(1, -1),
      f3w, f3b.reshape(1, -1))


_MESH = plsc.VectorSubcoreMesh(core_axis_name="c", subcore_axis_name="s")


def _sc_degrees(src2d, dst2d):
    @functools.partial(
        pl.kernel, mesh=_MESH,
        out_type=(jax.ShapeDtypeStruct((NC * N_PAD,), jnp.float32),
                  jax.ShapeDtypeStruct((NC * N_PAD,), jnp.float32)),
        scratch_types=[pltpu.VMEM((K, SZ), jnp.int32),
                       pltpu.VMEM((K, SZ), jnp.int32),
                       pltpu.VMEM((SZ,), jnp.float32),
                       pltpu.VMEM((NT,), jnp.float32),
                       pltpu.VMEM_SHARED((N_PAD,), jnp.float32),
                       pltpu.VMEM_SHARED((N_PAD,), jnp.float32)])
    def deg_kernel(src_hbm, dst_hbm, dego_hbm, degi_hbm,
                   src_v, dst_v, ones_v, zer_v, dego_sh, degi_sh):
        cid = lax.axis_index("c")
        sid = lax.axis_index("s")
        wid = sid * NC + cid

        def fill(i, _):
            zer_v[pl.ds(i * 16, 16)] = jnp.zeros((16,), jnp.float32)
            return 0
        lax.fori_loop(0, NT // 16, fill, 0)
        for c in range(SZ // 16):
            ones_v[pl.ds(c * 16, 16)] = jnp.full((16,), 1.0, jnp.float32)
        pltpu.sync_copy(zer_v, dego_sh.at[pl.ds(sid * NT, NT)])
        pltpu.sync_copy(zer_v, degi_sh.at[pl.ds(sid * NT, NT)])
        plsc.subcore_barrier()

        def outer(g, _):
            row0 = wid * RW + g * K
            pltpu.sync_copy(src_hbm.at[pl.ds(row0, K)], src_v)
            pltpu.sync_copy(dst_hbm.at[pl.ds(row0, K)], dst_v)
            for j in range(K):
                pltpu.sync_copy(ones_v, dego_sh.at[src_v.at[j]], add=True)
                pltpu.sync_copy(ones_v, degi_sh.at[dst_v.at[j]], add=True)
            return 0
        lax.fori_loop(0, NCH, outer, 0)
        plsc.subcore_barrier()

        sl = pl.ds(sid * NT, NT)
        osl = pl.ds(cid * N_PAD + sid * NT, NT)
        pltpu.sync_copy(dego_sh.at[sl], zer_v)
        pltpu.sync_copy(zer_v, dego_hbm.at[osl])
        pltpu.sync_copy(degi_sh.at[sl], zer_v)
        pltpu.sync_copy(zer_v, degi_hbm.at[osl])

    dego, degi = deg_kernel(src2d, dst2d)
    return dego.reshape(NC, N_PAD), degi.reshape(NC, N_PAD)


def _sc_mp(t, src2d, dst2d, ew_flat, w):
    zr = 112

    @functools.partial(
        pl.kernel, mesh=_MESH,
        out_type=jax.ShapeDtypeStruct((NC * N_PAD, w), jnp.float32),
        compiler_params=pltpu.CompilerParams(use_tc_tiling_on_sc=False),
        scratch_types=[pltpu.VMEM((K, SZ), jnp.int32),
                       pltpu.VMEM((K, SZ), jnp.int32),
                       pltpu.VMEM((K * SZ,), jnp.float32),
                       pltpu.VMEM((SZ, w), jnp.float32),
                       pltpu.VMEM((zr, w), jnp.float32),
                       pltpu.VMEM_SHARED((N_PAD, w), jnp.float32),
                       pltpu.SemaphoreType.DMA])
    def mp_kernel(t_hbm, src_hbm, dst_hbm, ew_hbm, out_hbm,
                  src_v, dst_v, ew_v, rows_v, zer_v, agg_sh, sem):
        cid = lax.axis_index("c")
        sid = lax.axis_index("s")
        wid = sid * NC + cid

        def fill(i, _):
            for c in range(w // 16):
                zer_v[i, pl.ds(c * 16, 16)] = jnp.zeros((16,), jnp.float32)
            return 0
        lax.fori_loop(0, zr, fill, 0)

        def zero(q, _):
            pltpu.sync_copy(zer_v, agg_sh.at[pl.ds(sid * NT + q * zr, zr)])
            return 0
        lax.fori_loop(0, NT // zr, zero, 0)
        plsc.subcore_barrier()

        def outer(g, _):
            row0 = wid * RW + g * K
            pltpu.sync_copy(src_hbm.at[pl.ds(row0, K)], src_v)
            pltpu.sync_copy(dst_hbm.at[pl.ds(row0, K)], dst_v)
            pltpu.sync_copy(ew_hbm.at[pl.ds(row0 * SZ, K * SZ)], ew_v)
            for j in range(K):
                pltpu.async_copy(t_hbm.at[src_v.at[j]], rows_v, sem).wait()

                def scale(b, _):
                    ewv = ew_v[pl.ds(j * SZ + b * 16, 16)]
                    for l in range(16):
                        s = ewv[l]
                        e = b * 16 + l
                        for c in range(w // 16):
                            csl = pl.ds(c * 16, 16)
                            rows_v[e, csl] = rows_v[e, csl] * s
                    return 0
                lax.fori_loop(0, SZ // 16, scale, 0)
                pltpu.sync_copy(rows_v, agg_sh.at[dst_v.at[j]], add=True)
            return 0
        lax.fori_loop(0, NCH, outer, 0)
        plsc.subcore_barrier()

        def wb(q, _):
            off = sid * NT + q * zr
            pltpu.sync_copy(agg_sh.at[pl.ds(off, zr)], zer_v)
            pltpu.sync_copy(zer_v, out_hbm.at[pl.ds(cid * N_PAD + off, zr)])
            return 0
        lax.fori_loop(0, NT // zr, wb, 0)

    return mp_kernel(t, src2d, dst2d, ew_flat).reshape(NC, N_PAD, w)


def kernel(x, edge_index, edge_weights, W1, b1, W2, b2, W3, b3, W4, b4,
           gamma, beta, fc1_W, fc1_b, fc2_W, fc2_b, fc3_W, fc3_b):
    src = edge_index[0].astype(jnp.int32)
    dst = edge_index[1].astype(jnp.int32)
    ew = edge_weights

    pad = jnp.full((E_PAD - E,), N, jnp.int32)
    src2d = jnp.concatenate([src, pad]).reshape(E_PAD // SZ, SZ)
    dst2d = jnp.concatenate([dst, pad]).reshape(E_PAD // SZ, SZ)
    ew_flat = jnp.concatenate([ew, jnp.zeros((E_PAD - E,), jnp.float32)])

    x_pad = jnp.zeros((N_PAD, 8), jnp.float32).at[:N, :6].set(x)
    w1_pad = jnp.zeros((8, 16), jnp.float32).at[:6].set(W1)

    dego, degi = _sc_degrees(src2d, dst2d)

    t1 = _tc_t1(dego, degi, x_pad, w1_pad)
    agg1 = _sc_mp(t1, src2d, dst2d, ew_flat, 16)
    t2 = _tc_mid(dego, degi, agg1, None, b1, 16, 16)
    agg2 = _sc_mp(t2, src2d, dst2d, ew_flat, 16)
    t3 = _tc_mid(dego, degi, agg2, W2, b2, 16, 32)
    agg3 = _sc_mp(t3, src2d, dst2d, ew_flat, 32)
    t4a, t4b = _tc_split(dego, degi, agg3, W3, b3)
    agg4a = _sc_mp(t4a, src2d, dst2d, ew_flat, 32)
    agg4b = _sc_mp(t4b, src2d, dst2d, ew_flat, 32)
    return _tc_head(dego, degi, agg4a, agg4b, W4, b4, gamma, beta,
                    fc1_W, fc1_b, fc2_W, fc2_b, fc3_W, fc3_b)


# speedup vs baseline: 9.7980x; 1.0742x over previous
import functools
import jax
import jax.numpy as jnp
import numpy as np
from jax import lax
from jax.experimental import pallas as pl
from jax.experimental.pallas import tpu as pltpu
from jax.experimental.pallas import tpu_sc as plsc

N = 50000
E = 1600000
N_PAD = 50176
R = 1792
NBLK = N_PAD // R

NC = 2
NS = 16
NW = NC * NS
SZ = 128
RW = 400
E_PAD = NW * RW * SZ
K = 16
NCH = RW // K
NT = N_PAD // NS
W16 = 16


def _lrelu(v, a):
    return jnp.where(v >= 0, v, a * v)


def _scales(dego, degi):
    dout = lax.rsqrt(jnp.maximum(dego[0] + dego[1], 1.0))
    din = lax.rsqrt(jnp.maximum(degi[0] + degi[1], 1.0))
    return dout, din


def _deg_spec():
    return pl.BlockSpec((2, R), lambda i: (0, i))


def _full(shape):
    return pl.BlockSpec(shape, lambda i: tuple(0 for _ in shape))


def _t1_body(dego_ref, degi_ref, x_ref, w1_ref, o_ref):
    dout, _ = _scales(dego_ref[...], degi_ref[...])
    o_ref[...] = jnp.dot(x_ref[...] * dout[:, None], w1_ref[...],
                         preferred_element_type=jnp.float32)


def _tc_t1(dego, degi, x_pad, w1):
    return pl.pallas_call(
        _t1_body,
        grid=(NBLK,),
        in_specs=[_deg_spec(), _deg_spec(),
                  pl.BlockSpec((R, 8), lambda i: (i, 0)),
                  _full((8, 16))],
        out_specs=pl.BlockSpec((R, 16), lambda i: (i, 0)),
        out_shape=jax.ShapeDtypeStruct((N_PAD, 16), jnp.float32),
    )(dego, degi, x_pad, w1)


def _dense_body(*refs, na, has_w, nout):
    dego_ref, degi_ref = refs[0], refs[1]
    aggs = refs[2:2 + na]
    pos = 2 + na
    if has_w:
        w_ref = refs[pos]
        pos += 1
    b_ref = refs[pos]
    outs = refs[pos + 1:]
    dout, din = _scales(dego_ref[...], degi_ref[...])
    acc = None
    for i in range(na):
        a = aggs[i][0] + aggs[i][1]
        if has_w:
            term = jnp.dot(a, w_ref[pl.ds(i * 16, 16), :],
                           preferred_element_type=jnp.float32)
        else:
            term = a
        acc = term if acc is None else acc + term
    h = _lrelu(acc * din[:, None] + b_ref[...], 0.01) * dout[:, None]
    for c in range(nout):
        outs[c][...] = h[:, c * 16:(c + 1) * 16]


def _tc_dense(dego, degi, aggs, w, b, nout):
    na = len(aggs)
    has_w = w is not None
    in_specs = [_deg_spec(), _deg_spec()]
    args = [dego, degi]
    for a in aggs:
        in_specs.append(pl.BlockSpec((2, R, 16), lambda i: (0, i, 0)))
        args.append(a)
    if has_w:
        in_specs.append(_full(w.shape))
        args.append(w)
    wout = nout * 16
    in_specs.append(_full((1, wout)))
    args.append(b.reshape(1, -1))
    return pl.pallas_call(
        functools.partial(_dense_body, na=na, has_w=has_w, nout=nout),
        grid=(NBLK,),
        in_specs=in_specs,
        out_specs=[pl.BlockSpec((R, 16), lambda i: (i, 0))] * nout,
        out_shape=[jax.ShapeDtypeStruct((N_PAD, 16), jnp.float32)] * nout,
    )(*args)


def _head_body(dego_ref, degi_ref, a0_ref, a1_ref, a2_ref, a3_ref,
               w4_ref, b4_ref, gb_ref,
               f1w_ref, f1b_ref, f2w_ref, f2b_ref, f3w_ref, f3b_ref,
               o_ref, acc_ref):
    i = pl.program_id(0)
    _, din = _scales(dego_ref[...], degi_ref[...])
    aggs = (a0_ref, a1_ref, a2_ref, a3_ref)
    h = None
    for c in range(4):
        term = jnp.dot(aggs[c][0] + aggs[c][1], w4_ref[pl.ds(c * 16, 16), :],
                       preferred_element_type=jnp.float32)
        h = term if h is None else h + term
    h = h * din[:, None] + b4_ref[...]
    h = h * gb_ref[0:1, :] + gb_ref[1:2, :]
    z = _lrelu(h, 0.01)
    rows = i * R + lax.broadcasted_iota(jnp.int32, (R, 1), 0)
    z = jnp.where(rows < N, z, 0.0)
    part = jnp.sum(z, axis=0, keepdims=True)

    @pl.when(i == 0)
    def _():
        acc_ref[...] = jnp.zeros_like(acc_ref)

    acc_ref[...] += part

    @pl.when(i == NBLK - 1)
    def _():
        g = acc_ref[...]
        o = _lrelu(jnp.dot(g, f1w_ref[...]) + f1b_ref[...], 0.1)
        o = _lrelu(jnp.dot(o, f2w_ref[...]) + f2b_ref[...], 0.1)
        o_ref[...] = jnp.dot(o, f3w_ref[...]) + f3b_ref[...]


def _tc_head(dego, degi, aggs, w4, b4, gamma, beta,
             f1w, f1b, f2w, f2b, f3w, f3b):
    gb = jnp.stack([gamma * np.float32(1.0 / np.sqrt(1.0 + 1e-5)), beta])
    agg_spec = pl.BlockSpec((2, R, 16), lambda i: (0, i, 0))
    return pl.pallas_call(
        _head_body,
        grid=(NBLK,),
        in_specs=[_deg_spec(), _deg_spec(),
                  agg_spec, agg_spec, agg_spec, agg_spec,
                  _full((64, 50)), _full((1, 50)), _full((2, 50)),
                  _full((50, 30)), _full((1, 30)),
                  _full((30, 10)), _full((1, 10)),
                  _full((10, 2)), _full((1, 2))],
        out_specs=pl.BlockSpec((1, 2), lambda i: (0, 0)),
        out_shape=jax.ShapeDtypeStruct((1, 2), jnp.float32),
        scratch_shapes=[pltpu.VMEM((1, 50), jnp.float32)],
        compiler_params=pltpu.CompilerParams(
            dimension_semantics=("arbitrary",)),
    )(dego, degi, *aggs, w4, b4.reshape(1, -1), gb,
      f1w, f1b.reshape(1, -1), f2w, f2b.reshape(1, -1),
      f3w, f3b.reshape(1, -1))


_MESH = plsc.VectorSubcoreMesh(core_axis_name="c", subcore_axis_name="s")


def _sc_degrees(src2d, dst2d):
    @functools.partial(
        pl.kernel, mesh=_MESH,
        out_type=(jax.ShapeDtypeStruct((NC * N_PAD,), jnp.float32),
                  jax.ShapeDtypeStruct((NC * N_PAD,), jnp.float32)),
        compiler_params=pltpu.CompilerParams(use_tc_tiling_on_sc=False),
        scratch_types=[pltpu.VMEM((K, SZ), jnp.int32),
                       pltpu.VMEM((K, SZ), jnp.int32),
                       pltpu.VMEM((SZ,), jnp.float32),
                       pltpu.VMEM((NT,), jnp.float32),
                       pltpu.VMEM_SHARED((N_PAD,), jnp.float32),
                       pltpu.VMEM_SHARED((N_PAD,), jnp.float32)])
    def deg_kernel(src_hbm, dst_hbm, dego_hbm, degi_hbm,
                   src_v, dst_v, ones_v, zer_v, dego_sh, degi_sh):
        cid = lax.axis_index("c")
        sid = lax.axis_index("s")
        wid = sid * NC + cid

        def fill(i, _):
            zer_v[pl.ds(i * 16, 16)] = jnp.zeros((16,), jnp.float32)
            return 0
        lax.fori_loop(0, NT // 16, fill, 0)
        for c in range(SZ // 16):
            ones_v[pl.ds(c * 16, 16)] = jnp.full((16,), 1.0, jnp.float32)
        pltpu.sync_copy(zer_v, dego_sh.at[pl.ds(sid * NT, NT)])
        pltpu.sync_copy(zer_v, degi_sh.at[pl.ds(sid * NT, NT)])
        plsc.subcore_barrier()

        def outer(g, _):
            row0 = wid * RW + g * K
            pltpu.sync_copy(src_hbm.at[pl.ds(row0, K)], src_v)
            pltpu.sync_copy(dst_hbm.at[pl.ds(row0, K)], dst_v)
            for j in range(K):
                pltpu.sync_copy(ones_v, dego_sh.at[src_v.at[j]], add=True)
                pltpu.sync_copy(ones_v, degi_sh.at[dst_v.at[j]], add=True)
            return 0
        lax.fori_loop(0, NCH, outer, 0)
        plsc.subcore_barrier()

        sl = pl.ds(sid * NT, NT)
        osl = pl.ds(cid * N_PAD + sid * NT, NT)
        pltpu.sync_copy(dego_sh.at[sl], zer_v)
        pltpu.sync_copy(zer_v, dego_hbm.at[osl])
        pltpu.sync_copy(degi_sh.at[sl], zer_v)
        pltpu.sync_copy(zer_v, degi_hbm.at[osl])

    dego, degi = deg_kernel(src2d, dst2d)
    return dego.reshape(NC, N_PAD), degi.reshape(NC, N_PAD)


def _sc_mp(t, src2d, dst2d, ew_flat):
    zr = 112

    @functools.partial(
        pl.kernel, mesh=_MESH,
        out_type=jax.ShapeDtypeStruct((NC * N_PAD, W16), jnp.float32),
        compiler_params=pltpu.CompilerParams(use_tc_tiling_on_sc=False),
        scratch_types=[pltpu.VMEM((K, SZ), jnp.int32),
                       pltpu.VMEM((K, SZ), jnp.int32),
                       pltpu.VMEM((K * SZ,), jnp.float32),
                       pltpu.VMEM((2, SZ, W16), jnp.float32),
                       pltpu.VMEM((zr, W16), jnp.float32),
                       pltpu.VMEM_SHARED((N_PAD, W16), jnp.float32),
                       pltpu.SemaphoreType.DMA,
                       pltpu.SemaphoreType.DMA])
    def mp_kernel(t_hbm, src_hbm, dst_hbm, ew_hbm, out_hbm,
                  src_v, dst_v, ew_v, rows_v, zer_v, agg_sh, gsem0, gsem1):
        cid = lax.axis_index("c")
        sid = lax.axis_index("s")
        wid = sid * NC + cid

        def fill(i, _):
            zer_v[i, :] = jnp.zeros((16,), jnp.float32)
            return 0
        lax.fori_loop(0, zr, fill, 0)

        def zero(q, _):
            pltpu.sync_copy(zer_v, agg_sh.at[pl.ds(sid * NT + q * zr, zr)])
            return 0
        lax.fori_loop(0, NT // zr, zero, 0)
        plsc.subcore_barrier()

        def outer(g, _):
            row0 = wid * RW + g * K
            pltpu.sync_copy(src_hbm.at[pl.ds(row0, K)], src_v)
            pltpu.sync_copy(dst_hbm.at[pl.ds(row0, K)], dst_v)
            pltpu.sync_copy(ew_hbm.at[pl.ds(row0 * SZ, K * SZ)], ew_v)
            sems = (gsem0, gsem1)
            cps = [pltpu.async_copy(t_hbm.at[src_v.at[0]], rows_v.at[0],
                                    gsem0)]
            for j in range(K):
                b = j % 2
                if j + 1 < K:
                    cps.append(pltpu.async_copy(t_hbm.at[src_v.at[j + 1]],
                                                rows_v.at[1 - b],
                                                sems[1 - b]))
                cps[j].wait()

                def scale(bk, _):
                    ewv = ew_v[pl.ds(j * SZ + bk * 16, 16)]
                    for l in range(16):
                        s = ewv[l]
                        e = bk * 16 + l
                        rows_v[b, e, :] = rows_v[b, e, :] * s
                    return 0
                lax.fori_loop(0, SZ // 16, scale, 0)
                pltpu.sync_copy(rows_v.at[b], agg_sh.at[dst_v.at[j]],
                                add=True)
            return 0
        lax.fori_loop(0, NCH, outer, 0)
        plsc.subcore_barrier()

        def wb(q, _):
            off = sid * NT + q * zr
            pltpu.sync_copy(agg_sh.at[pl.ds(off, zr)], zer_v)
            pltpu.sync_copy(zer_v, out_hbm.at[pl.ds(cid * N_PAD + off, zr)])
            return 0
        lax.fori_loop(0, NT // zr, wb, 0)

    return mp_kernel(t, src2d, dst2d, ew_flat).reshape(NC, N_PAD, W16)


def kernel(x, edge_index, edge_weights, W1, b1, W2, b2, W3, b3, W4, b4,
           gamma, beta, fc1_W, fc1_b, fc2_W, fc2_b, fc3_W, fc3_b):
    src = edge_index[0].astype(jnp.int32)
    dst = edge_index[1].astype(jnp.int32)
    ew = edge_weights

    pad = jnp.full((E_PAD - E,), N, jnp.int32)
    src2d = jnp.concatenate([src, pad]).reshape(E_PAD // SZ, SZ)
    dst2d = jnp.concatenate([dst, pad]).reshape(E_PAD // SZ, SZ)
    ew_flat = jnp.concatenate([ew, jnp.zeros((E_PAD - E,), jnp.float32)])

    x_pad = jnp.zeros((N_PAD, 8), jnp.float32).at[:N, :6].set(x)
    w1_pad = jnp.zeros((8, 16), jnp.float32).at[:6].set(W1)

    dego, degi = _sc_degrees(src2d, dst2d)

    t1 = _tc_t1(dego, degi, x_pad, w1_pad)
    agg1 = _sc_mp(t1, src2d, dst2d, ew_flat)
    (t2,) = _tc_dense(dego, degi, [agg1], None, b1, 1)
    agg2 = _sc_mp(t2, src2d, dst2d, ew_flat)
    t3a, t3b = _tc_dense(dego, degi, [agg2], W2, b2, 2)
    agg3a = _sc_mp(t3a, src2d, dst2d, ew_flat)
    agg3b = _sc_mp(t3b, src2d, dst2d, ew_flat)
    t4 = _tc_dense(dego, degi, [agg3a, agg3b], W3, b3, 4)
    agg4 = [_sc_mp(tc, src2d, dst2d, ew_flat) for tc in t4]
    return _tc_head(dego, degi, agg4, W4, b4, gamma, beta,
                    fc1_W, fc1_b, fc2_W, fc2_b, fc3_W, fc3_b)


# speedup vs baseline: 11.7962x; 1.2039x over previous
import functools
import jax
import jax.numpy as jnp
import numpy as np
from jax import lax
from jax.experimental import pallas as pl
from jax.experimental.pallas import tpu as pltpu
from jax.experimental.pallas import tpu_sc as plsc

N = 50000
E = 1600000
N_PAD = 50176
R = 1792
NBLK = N_PAD // R

NC = 2
NS = 16
NW = NC * NS
SZ = 128
RW = 400
E_PAD = NW * RW * SZ
K = 16
NCH = RW // K
NT = N_PAD // NS
W16 = 16


def _lrelu(v, a):
    return jnp.where(v >= 0, v, a * v)


def _scales(dego, degi):
    dout = lax.rsqrt(jnp.maximum(dego[0] + dego[1], 1.0))
    din = lax.rsqrt(jnp.maximum(degi[0] + degi[1], 1.0))
    return dout, din


def _deg_spec():
    return pl.BlockSpec((2, R), lambda i: (0, i))


def _full(shape):
    return pl.BlockSpec(shape, lambda i: tuple(0 for _ in shape))


def _t1_body(dego_ref, degi_ref, x_ref, w1_ref, o_ref):
    dout, _ = _scales(dego_ref[...], degi_ref[...])
    o_ref[...] = jnp.dot(x_ref[...] * dout[:, None], w1_ref[...],
                         preferred_element_type=jnp.float32)


def _tc_t1(dego, degi, x_pad, w1):
    return pl.pallas_call(
        _t1_body,
        grid=(NBLK,),
        in_specs=[_deg_spec(), _deg_spec(),
                  pl.BlockSpec((R, 8), lambda i: (i, 0)),
                  _full((8, 16))],
        out_specs=pl.BlockSpec((R, 16), lambda i: (i, 0)),
        out_shape=jax.ShapeDtypeStruct((N_PAD, 16), jnp.float32),
    )(dego, degi, x_pad, w1)


def _dense_body(*refs, na, aw, has_w, nout, ow):
    dego_ref, degi_ref = refs[0], refs[1]
    aggs = refs[2:2 + na]
    pos = 2 + na
    if has_w:
        w_ref = refs[pos]
        pos += 1
    b_ref = refs[pos]
    outs = refs[pos + 1:]
    dout, din = _scales(dego_ref[...], degi_ref[...])
    acc = None
    for i in range(na):
        a = aggs[i][0] + aggs[i][1]
        if has_w:
            term = jnp.dot(a, w_ref[pl.ds(i * aw, aw), :],
                           preferred_element_type=jnp.float32)
        else:
            term = a
        acc = term if acc is None else acc + term
    h = _lrelu(acc * din[:, None] + b_ref[...], 0.01) * dout[:, None]
    for c in range(nout):
        outs[c][...] = h[:, c * ow:(c + 1) * ow]


def _tc_dense(dego, degi, aggs, aw, w, b, nout, ow):
    na = len(aggs)
    has_w = w is not None
    in_specs = [_deg_spec(), _deg_spec()]
    args = [dego, degi]
    for a in aggs:
        in_specs.append(pl.BlockSpec((2, R, aw), lambda i: (0, i, 0)))
        args.append(a)
    if has_w:
        in_specs.append(_full(w.shape))
        args.append(w)
    in_specs.append(_full((1, nout * ow)))
    args.append(b.reshape(1, -1))
    return pl.pallas_call(
        functools.partial(_dense_body, na=na, aw=aw, has_w=has_w,
                          nout=nout, ow=ow),
        grid=(NBLK,),
        in_specs=in_specs,
        out_specs=[pl.BlockSpec((R, ow), lambda i: (i, 0))] * nout,
        out_shape=[jax.ShapeDtypeStruct((N_PAD, ow), jnp.float32)] * nout,
    )(*args)


def _head_body(dego_ref, degi_ref, a0_ref, a1_ref,
               w4_ref, b4_ref, gb_ref,
               f1w_ref, f1b_ref, f2w_ref, f2b_ref, f3w_ref, f3b_ref,
               o_ref, acc_ref):
    i = pl.program_id(0)
    _, din = _scales(dego_ref[...], degi_ref[...])
    aggs = (a0_ref, a1_ref)
    h = None
    for c in range(2):
        term = jnp.dot(aggs[c][0] + aggs[c][1], w4_ref[pl.ds(c * 32, 32), :],
                       preferred_element_type=jnp.float32)
        h = term if h is None else h + term
    h = h * din[:, None] + b4_ref[...]
    h = h * gb_ref[0:1, :] + gb_ref[1:2, :]
    z = _lrelu(h, 0.01)
    rows = i * R + lax.broadcasted_iota(jnp.int32, (R, 1), 0)
    z = jnp.where(rows < N, z, 0.0)
    part = jnp.sum(z, axis=0, keepdims=True)

    @pl.when(i == 0)
    def _():
        acc_ref[...] = jnp.zeros_like(acc_ref)

    acc_ref[...] += part

    @pl.when(i == NBLK - 1)
    def _():
        g = acc_ref[...]
        o = _lrelu(jnp.dot(g, f1w_ref[...]) + f1b_ref[...], 0.1)
        o = _lrelu(jnp.dot(o, f2w_ref[...]) + f2b_ref[...], 0.1)
        o_ref[...] = jnp.dot(o, f3w_ref[...]) + f3b_ref[...]


def _tc_head(dego, degi, aggs, w4, b4, gamma, beta,
             f1w, f1b, f2w, f2b, f3w, f3b):
    gb = jnp.stack([gamma * np.float32(1.0 / np.sqrt(1.0 + 1e-5)), beta])
    agg_spec = pl.BlockSpec((2, R, 32), lambda i: (0, i, 0))
    return pl.pallas_call(
        _head_body,
        grid=(NBLK,),
        in_specs=[_deg_spec(), _deg_spec(),
                  agg_spec, agg_spec,
                  _full((64, 50)), _full((1, 50)), _full((2, 50)),
                  _full((50, 30)), _full((1, 30)),
                  _full((30, 10)), _full((1, 10)),
                  _full((10, 2)), _full((1, 2))],
        out_specs=pl.BlockSpec((1, 2), lambda i: (0, 0)),
        out_shape=jax.ShapeDtypeStruct((1, 2), jnp.float32),
        scratch_shapes=[pltpu.VMEM((1, 50), jnp.float32)],
        compiler_params=pltpu.CompilerParams(
            dimension_semantics=("arbitrary",)),
    )(dego, degi, *aggs, w4, b4.reshape(1, -1), gb,
      f1w, f1b.reshape(1, -1), f2w, f2b.reshape(1, -1),
      f3w, f3b.reshape(1, -1))


_MESH = plsc.VectorSubcoreMesh(core_axis_name="c", subcore_axis_name="s")


def _sc_degrees(src2d, dst2d):
    @functools.partial(
        pl.kernel, mesh=_MESH,
        out_type=(jax.ShapeDtypeStruct((NC * N_PAD,), jnp.float32),
                  jax.ShapeDtypeStruct((NC * N_PAD,), jnp.float32)),
        compiler_params=pltpu.CompilerParams(use_tc_tiling_on_sc=False),
        scratch_types=[pltpu.VMEM((K, SZ), jnp.int32),
                       pltpu.VMEM((K, SZ), jnp.int32),
                       pltpu.VMEM((SZ,), jnp.float32),
                       pltpu.VMEM((NT,), jnp.float32),
                       pltpu.VMEM_SHARED((N_PAD,), jnp.float32),
                       pltpu.VMEM_SHARED((N_PAD,), jnp.float32)])
    def deg_kernel(src_hbm, dst_hbm, dego_hbm, degi_hbm,
                   src_v, dst_v, ones_v, zer_v, dego_sh, degi_sh):
        cid = lax.axis_index("c")
        sid = lax.axis_index("s")
        wid = sid * NC + cid

        def fill(i, _):
            zer_v[pl.ds(i * 16, 16)] = jnp.zeros((16,), jnp.float32)
            return 0
        lax.fori_loop(0, NT // 16, fill, 0)
        for c in range(SZ // 16):
            ones_v[pl.ds(c * 16, 16)] = jnp.full((16,), 1.0, jnp.float32)
        pltpu.sync_copy(zer_v, dego_sh.at[pl.ds(sid * NT, NT)])
        pltpu.sync_copy(zer_v, degi_sh.at[pl.ds(sid * NT, NT)])
        plsc.subcore_barrier()

        def outer(g, _):
            row0 = wid * RW + g * K
            pltpu.sync_copy(src_hbm.at[pl.ds(row0, K)], src_v)
            pltpu.sync_copy(dst_hbm.at[pl.ds(row0, K)], dst_v)
            for j in range(K):
                pltpu.sync_copy(ones_v, dego_sh.at[src_v.at[j]], add=True)
                pltpu.sync_copy(ones_v, degi_sh.at[dst_v.at[j]], add=True)
            return 0
        lax.fori_loop(0, NCH, outer, 0)
        plsc.subcore_barrier()

        sl = pl.ds(sid * NT, NT)
        osl = pl.ds(cid * N_PAD + sid * NT, NT)
        pltpu.sync_copy(dego_sh.at[sl], zer_v)
        pltpu.sync_copy(zer_v, dego_hbm.at[osl])
        pltpu.sync_copy(degi_sh.at[sl], zer_v)
        pltpu.sync_copy(zer_v, degi_hbm.at[osl])

    dego, degi = deg_kernel(src2d, dst2d)
    return dego.reshape(NC, N_PAD), degi.reshape(NC, N_PAD)


def _sc_mp(t, src2d, dst2d, ew_flat, w):
    zr = 112

    @functools.partial(
        pl.kernel, mesh=_MESH,
        out_type=jax.ShapeDtypeStruct((NC * N_PAD, w), jnp.float32),
        compiler_params=pltpu.CompilerParams(use_tc_tiling_on_sc=False),
        scratch_types=[pltpu.VMEM((K, SZ), jnp.int32),
                       pltpu.VMEM((K, SZ), jnp.int32),
                       pltpu.VMEM((K * SZ,), jnp.float32),
                       pltpu.VMEM((2, SZ, w), jnp.float32),
                       pltpu.VMEM((zr, w), jnp.float32),
                       pltpu.VMEM_SHARED((N_PAD, w), jnp.float32),
                       pltpu.SemaphoreType.DMA,
                       pltpu.SemaphoreType.DMA])
    def mp_kernel(t_hbm, src_hbm, dst_hbm, ew_hbm, out_hbm,
                  src_v, dst_v, ew_v, rows_v, zer_v, agg_sh, gsem0, gsem1):
        cid = lax.axis_index("c")
        sid = lax.axis_index("s")
        wid = sid * NC + cid

        def fill(i, _):
            for c in range(w // 16):
                zer_v[i, pl.ds(c * 16, 16)] = jnp.zeros((16,), jnp.float32)
            return 0
        lax.fori_loop(0, zr, fill, 0)

        def zero(q, _):
            pltpu.sync_copy(zer_v, agg_sh.at[pl.ds(sid * NT + q * zr, zr)])
            return 0
        lax.fori_loop(0, NT // zr, zero, 0)
        plsc.subcore_barrier()

        def outer(g, _):
            row0 = wid * RW + g * K
            pltpu.sync_copy(src_hbm.at[pl.ds(row0, K)], src_v)
            pltpu.sync_copy(dst_hbm.at[pl.ds(row0, K)], dst_v)
            pltpu.sync_copy(ew_hbm.at[pl.ds(row0 * SZ, K * SZ)], ew_v)
            sems = (gsem0, gsem1)
            cps = [pltpu.async_copy(t_hbm.at[src_v.at[0]], rows_v.at[0],
                                    gsem0)]
            for j in range(K):
                b = j % 2
                if j + 1 < K:
                    cps.append(pltpu.async_copy(t_hbm.at[src_v.at[j + 1]],
                                                rows_v.at[1 - b],
                                                sems[1 - b]))
                cps[j].wait()

                def scale(bk, _):
                    ewv = ew_v[pl.ds(j * SZ + bk * 16, 16)]
                    for l in range(16):
                        s = ewv[l]
                        e = bk * 16 + l
                        for c in range(w // 16):
                            csl = pl.ds(c * 16, 16)
                            rows_v[b, e, csl] = rows_v[b, e, csl] * s
                    return 0
                lax.fori_loop(0, SZ // 16, scale, 0)
                pltpu.sync_copy(rows_v.at[b], agg_sh.at[dst_v.at[j]],
                                add=True)
            return 0
        lax.fori_loop(0, NCH, outer, 0)
        plsc.subcore_barrier()

        def wb(q, _):
            off = sid * NT + q * zr
            pltpu.sync_copy(agg_sh.at[pl.ds(off, zr)], zer_v)
            pltpu.sync_copy(zer_v, out_hbm.at[pl.ds(cid * N_PAD + off, zr)])
            return 0
        lax.fori_loop(0, NT // zr, wb, 0)

    return mp_kernel(t, src2d, dst2d, ew_flat).reshape(NC, N_PAD, w)


def kernel(x, edge_index, edge_weights, W1, b1, W2, b2, W3, b3, W4, b4,
           gamma, beta, fc1_W, fc1_b, fc2_W, fc2_b, fc3_W, fc3_b):
    src = edge_index[0].astype(jnp.int32)
    dst = edge_index[1].astype(jnp.int32)
    ew = edge_weights

    pad = jnp.full((E_PAD - E,), N, jnp.int32)
    src2d = jnp.concatenate([src, pad]).reshape(E_PAD // SZ, SZ)
    dst2d = jnp.concatenate([dst, pad]).reshape(E_PAD // SZ, SZ)
    ew_flat = jnp.concatenate([ew, jnp.zeros((E_PAD - E,), jnp.float32)])

    x_pad = jnp.zeros((N_PAD, 8), jnp.float32).at[:N, :6].set(x)
    w1_pad = jnp.zeros((8, 16), jnp.float32).at[:6].set(W1)

    dego, degi = _sc_degrees(src2d, dst2d)

    t1 = _tc_t1(dego, degi, x_pad, w1_pad)
    agg1 = _sc_mp(t1, src2d, dst2d, ew_flat, 16)
    (t2,) = _tc_dense(dego, degi, [agg1], 16, None, b1, 1, 16)
    agg2 = _sc_mp(t2, src2d, dst2d, ew_flat, 16)
    (t3,) = _tc_dense(dego, degi, [agg2], 16, W2, b2, 1, 32)
    agg3 = _sc_mp(t3, src2d, dst2d, ew_flat, 32)
    t4a, t4b = _tc_dense(dego, degi, [agg3], 32, W3, b3, 2, 32)
    agg4a = _sc_mp(t4a, src2d, dst2d, ew_flat, 32)
    agg4b = _sc_mp(t4b, src2d, dst2d, ew_flat, 32)
    return _tc_head(dego, degi, [agg4a, agg4b], W4, b4, gamma, beta,
                    fc1_W, fc1_b, fc2_W, fc2_b, fc3_W, fc3_b)


# speedup vs baseline: 12.5709x; 1.0657x over previous
import functools
import jax
import jax.numpy as jnp
import numpy as np
from jax import lax
from jax.experimental import pallas as pl
from jax.experimental.pallas import tpu as pltpu
from jax.experimental.pallas import tpu_sc as plsc

N = 50000
E = 1600000
N_PAD = 50176
R = 1792
NBLK = N_PAD // R

NC = 2
NS = 16
NW = NC * NS
SZ = 128
RW = 400
E_PAD = NW * RW * SZ
K = 16
NCH = RW // K
NT = N_PAD // NS
W16 = 16


def _lrelu(v, a):
    return jnp.where(v >= 0, v, a * v)


def _scales(dego, degi):
    dout = lax.rsqrt(jnp.maximum(dego[0] + dego[1], 1.0))
    din = lax.rsqrt(jnp.maximum(degi[0] + degi[1], 1.0))
    return dout, din


def _deg_spec():
    return pl.BlockSpec((2, R), lambda i: (0, i))


def _full(shape):
    return pl.BlockSpec(shape, lambda i: tuple(0 for _ in shape))


def _t1_body(dego_ref, degi_ref, x_ref, w1_ref, o_ref):
    dout, _ = _scales(dego_ref[...], degi_ref[...])
    o_ref[...] = jnp.dot(x_ref[...] * dout[:, None], w1_ref[...],
                         preferred_element_type=jnp.float32)


def _tc_t1(dego, degi, x_pad, w1):
    return pl.pallas_call(
        _t1_body,
        grid=(NBLK,),
        in_specs=[_deg_spec(), _deg_spec(),
                  pl.BlockSpec((R, 8), lambda i: (i, 0)),
                  _full((8, 16))],
        out_specs=pl.BlockSpec((R, 16), lambda i: (i, 0)),
        out_shape=jax.ShapeDtypeStruct((N_PAD, 16), jnp.float32),
    )(dego, degi, x_pad, w1)


def _dense_body(*refs, na, aw, has_w, nout, ow):
    dego_ref, degi_ref = refs[0], refs[1]
    aggs = refs[2:2 + na]
    pos = 2 + na
    if has_w:
        w_ref = refs[pos]
        pos += 1
    b_ref = refs[pos]
    outs = refs[pos + 1:]
    dout, din = _scales(dego_ref[...], degi_ref[...])
    acc = None
    for i in range(na):
        a = aggs[i][0] + aggs[i][1]
        if has_w:
            term = jnp.dot(a, w_ref[pl.ds(i * aw, aw), :],
                           preferred_element_type=jnp.float32)
        else:
            term = a
        acc = term if acc is None else acc + term
    h = _lrelu(acc * din[:, None] + b_ref[...], 0.01) * dout[:, None]
    for c in range(nout):
        outs[c][...] = h[:, c * ow:(c + 1) * ow]


def _tc_dense(dego, degi, aggs, aw, w, b, nout, ow):
    na = len(aggs)
    has_w = w is not None
    in_specs = [_deg_spec(), _deg_spec()]
    args = [dego, degi]
    for a in aggs:
        in_specs.append(pl.BlockSpec((2, R, aw), lambda i: (0, i, 0)))
        args.append(a)
    if has_w:
        in_specs.append(_full(w.shape))
        args.append(w)
    in_specs.append(_full((1, nout * ow)))
    args.append(b.reshape(1, -1))
    return pl.pallas_call(
        functools.partial(_dense_body, na=na, aw=aw, has_w=has_w,
                          nout=nout, ow=ow),
        grid=(NBLK,),
        in_specs=in_specs,
        out_specs=[pl.BlockSpec((R, ow), lambda i: (i, 0))] * nout,
        out_shape=[jax.ShapeDtypeStruct((N_PAD, ow), jnp.float32)] * nout,
    )(*args)


def _head_body(dego_ref, degi_ref, a0_ref, a1_ref,
               w4_ref, b4_ref, gb_ref,
               f1w_ref, f1b_ref, f2w_ref, f2b_ref, f3w_ref, f3b_ref,
               o_ref, acc_ref):
    i = pl.program_id(0)
    _, din = _scales(dego_ref[...], degi_ref[...])
    aggs = (a0_ref, a1_ref)
    h = None
    for c in range(2):
        term = jnp.dot(aggs[c][0] + aggs[c][1], w4_ref[pl.ds(c * 32, 32), :],
                       preferred_element_type=jnp.float32)
        h = term if h is None else h + term
    h = h * din[:, None] + b4_ref[...]
    h = h * gb_ref[0:1, :] + gb_ref[1:2, :]
    z = _lrelu(h, 0.01)
    rows = i * R + lax.broadcasted_iota(jnp.int32, (R, 1), 0)
    z = jnp.where(rows < N, z, 0.0)
    part = jnp.sum(z, axis=0, keepdims=True)

    @pl.when(i == 0)
    def _():
        acc_ref[...] = jnp.zeros_like(acc_ref)

    acc_ref[...] += part

    @pl.when(i == NBLK - 1)
    def _():
        g = acc_ref[...]
        o = _lrelu(jnp.dot(g, f1w_ref[...]) + f1b_ref[...], 0.1)
        o = _lrelu(jnp.dot(o, f2w_ref[...]) + f2b_ref[...], 0.1)
        o_ref[...] = jnp.dot(o, f3w_ref[...]) + f3b_ref[...]


def _tc_head(dego, degi, aggs, w4, b4, gamma, beta,
             f1w, f1b, f2w, f2b, f3w, f3b):
    gb = jnp.stack([gamma * np.float32(1.0 / np.sqrt(1.0 + 1e-5)), beta])
    agg_spec = pl.BlockSpec((2, R, 32), lambda i: (0, i, 0))
    return pl.pallas_call(
        _head_body,
        grid=(NBLK,),
        in_specs=[_deg_spec(), _deg_spec(),
                  agg_spec, agg_spec,
                  _full((64, 50)), _full((1, 50)), _full((2, 50)),
                  _full((50, 30)), _full((1, 30)),
                  _full((30, 10)), _full((1, 10)),
                  _full((10, 2)), _full((1, 2))],
        out_specs=pl.BlockSpec((1, 2), lambda i: (0, 0)),
        out_shape=jax.ShapeDtypeStruct((1, 2), jnp.float32),
        scratch_shapes=[pltpu.VMEM((1, 50), jnp.float32)],
        compiler_params=pltpu.CompilerParams(
            dimension_semantics=("arbitrary",)),
    )(dego, degi, *aggs, w4, b4.reshape(1, -1), gb,
      f1w, f1b.reshape(1, -1), f2w, f2b.reshape(1, -1),
      f3w, f3b.reshape(1, -1))


_MESH = plsc.VectorSubcoreMesh(core_axis_name="c", subcore_axis_name="s")


def _sc_degrees(src2d, dst2d):
    @functools.partial(
        pl.kernel, mesh=_MESH,
        out_type=(jax.ShapeDtypeStruct((NC * N_PAD,), jnp.float32),
                  jax.ShapeDtypeStruct((NC * N_PAD,), jnp.float32)),
        compiler_params=pltpu.CompilerParams(use_tc_tiling_on_sc=False),
        scratch_types=[pltpu.VMEM((K, SZ), jnp.int32),
                       pltpu.VMEM((K, SZ), jnp.int32),
                       pltpu.VMEM((SZ,), jnp.float32),
                       pltpu.VMEM((NT,), jnp.float32),
                       pltpu.VMEM_SHARED((N_PAD,), jnp.float32),
                       pltpu.VMEM_SHARED((N_PAD,), jnp.float32)])
    def deg_kernel(src_hbm, dst_hbm, dego_hbm, degi_hbm,
                   src_v, dst_v, ones_v, zer_v, dego_sh, degi_sh):
        cid = lax.axis_index("c")
        sid = lax.axis_index("s")
        wid = sid * NC + cid

        def fill(i, _):
            zer_v[pl.ds(i * 16, 16)] = jnp.zeros((16,), jnp.float32)
            return 0
        lax.fori_loop(0, NT // 16, fill, 0)
        for c in range(SZ // 16):
            ones_v[pl.ds(c * 16, 16)] = jnp.full((16,), 1.0, jnp.float32)
        pltpu.sync_copy(zer_v, dego_sh.at[pl.ds(sid * NT, NT)])
        pltpu.sync_copy(zer_v, degi_sh.at[pl.ds(sid * NT, NT)])
        plsc.subcore_barrier()

        def outer(g, _):
            row0 = wid * RW + g * K
            pltpu.sync_copy(src_hbm.at[pl.ds(row0, K)], src_v)
            pltpu.sync_copy(dst_hbm.at[pl.ds(row0, K)], dst_v)
            for j in range(K):
                pltpu.sync_copy(ones_v, dego_sh.at[src_v.at[j]], add=True)
                pltpu.sync_copy(ones_v, degi_sh.at[dst_v.at[j]], add=True)
            return 0
        lax.fori_loop(0, NCH, outer, 0)
        plsc.subcore_barrier()

        sl = pl.ds(sid * NT, NT)
        osl = pl.ds(cid * N_PAD + sid * NT, NT)
        pltpu.sync_copy(dego_sh.at[sl], zer_v)
        pltpu.sync_copy(zer_v, dego_hbm.at[osl])
        pltpu.sync_copy(degi_sh.at[sl], zer_v)
        pltpu.sync_copy(zer_v, degi_hbm.at[osl])

    dego, degi = deg_kernel(src2d, dst2d)
    return dego.reshape(NC, N_PAD), degi.reshape(NC, N_PAD)


def _sc_mp(t, src2d, dst2d, ew_flat, w):
    zr = 56

    @functools.partial(
        pl.kernel, mesh=_MESH,
        out_type=jax.ShapeDtypeStruct((NC * N_PAD, w), jnp.float32),
        compiler_params=pltpu.CompilerParams(use_tc_tiling_on_sc=False),
        scratch_types=[pltpu.VMEM((K, SZ), jnp.int32),
                       pltpu.VMEM((K, SZ), jnp.int32),
                       pltpu.VMEM((K, SZ), jnp.int32),
                       pltpu.VMEM((K, SZ), jnp.int32),
                       pltpu.VMEM((K * SZ,), jnp.float32),
                       pltpu.VMEM((K * SZ,), jnp.float32),
                       pltpu.VMEM((4, SZ, w), jnp.float32),
                       pltpu.VMEM((zr, w), jnp.float32),
                       pltpu.VMEM_SHARED((N_PAD, w), jnp.float32),
                       pltpu.SemaphoreType.DMA,
                       pltpu.SemaphoreType.DMA,
                       pltpu.SemaphoreType.DMA,
                       pltpu.SemaphoreType.DMA,
                       pltpu.SemaphoreType.DMA,
                       pltpu.SemaphoreType.DMA,
                       pltpu.SemaphoreType.DMA,
                       pltpu.SemaphoreType.DMA,
                       pltpu.SemaphoreType.DMA,
                       pltpu.SemaphoreType.DMA])
    def mp_kernel(t_hbm, src_hbm, dst_hbm, ew_hbm, out_hbm,
                  src_v0, src_v1, dst_v0, dst_v1, ew_v0, ew_v1,
                  rows_v, zer_v, agg_sh,
                  gs0, gs1, gs2, gs3, ss0, ss1, ss2, ss3, is0, is1):
        cid = lax.axis_index("c")
        sid = lax.axis_index("s")
        wid = sid * NC + cid
        gsem = (gs0, gs1, gs2, gs3)
        ssem = (ss0, ss1, ss2, ss3)
        isem = (is0, is1)
        srcs = (src_v0, src_v1)
        dsts = (dst_v0, dst_v1)
        ews = (ew_v0, ew_v1)

        def fill(i, _):
            for c in range(w // 16):
                zer_v[i, pl.ds(c * 16, 16)] = jnp.zeros((16,), jnp.float32)
            return 0
        lax.fori_loop(0, zr, fill, 0)

        def zero(q, _):
            pltpu.sync_copy(zer_v, agg_sh.at[pl.ds(sid * NT + q * zr, zr)])
            return 0
        lax.fori_loop(0, NT // zr, zero, 0)
        plsc.subcore_barrier()

        def fire_idx(gch, par):
            row0 = wid * RW + gch * K
            pltpu.async_copy(src_hbm.at[pl.ds(row0, K)], srcs[par],
                             isem[par])
            pltpu.async_copy(dst_hbm.at[pl.ds(row0, K)], dsts[par],
                             isem[par])
            pltpu.async_copy(ew_hbm.at[pl.ds(row0 * SZ, K * SZ)], ews[par],
                             isem[par])

        def wait_idx(par):
            pltpu.make_async_copy(src_hbm.at[pl.ds(0, K)], srcs[par],
                                  isem[par]).wait()
            pltpu.make_async_copy(dst_hbm.at[pl.ds(0, K)], dsts[par],
                                  isem[par]).wait()
            pltpu.make_async_copy(ew_hbm.at[pl.ds(0, K * SZ)], ews[par],
                                  isem[par]).wait()

        def chunk(gch, par, prefetch):
            wait_idx(par)
            if prefetch:
                fire_idx(gch + 1, 1 - par)
            src_v, dst_v, ew_v = srcs[par], dsts[par], ews[par]
            gcps = [pltpu.async_copy(t_hbm.at[src_v.at[jj]], rows_v.at[jj],
                                     gsem[jj]) for jj in range(2)]
            scps = []
            for j in range(K):
                b = j % 4
                if j >= 2:
                    scps[j - 2].wait()
                if j + 2 < K:
                    nb = (j + 2) % 4
                    gcps.append(pltpu.async_copy(
                        t_hbm.at[src_v.at[j + 2]], rows_v.at[nb], gsem[nb]))
                gcps[j].wait()

                def scale(bk, _):
                    ewv = ew_v[pl.ds(j * SZ + bk * 16, 16)]
                    for l in range(16):
                        s = ewv[l]
                        e = bk * 16 + l
                        for c in range(w // 16):
                            csl = pl.ds(c * 16, 16)
                            rows_v[b, e, csl] = rows_v[b, e, csl] * s
                    return 0
                lax.fori_loop(0, SZ // 16, scale, 0)
                scps.append(pltpu.async_copy(rows_v.at[b],
                                             agg_sh.at[dst_v.at[j]],
                                             ssem[b], add=True))
            scps[K - 2].wait()
            scps[K - 1].wait()

        fire_idx(0, 0)

        def outer(g2, _):
            chunk(2 * g2, 0, True)
            chunk(2 * g2 + 1, 1, True)
            return 0
        lax.fori_loop(0, NCH // 2, outer, 0)
        chunk(NCH - 1, 0, False)
        plsc.subcore_barrier()

        def wb(q, _):
            off = sid * NT + q * zr
            pltpu.sync_copy(agg_sh.at[pl.ds(off, zr)], zer_v)
            pltpu.sync_copy(zer_v, out_hbm.at[pl.ds(cid * N_PAD + off, zr)])
            return 0
        lax.fori_loop(0, NT // zr, wb, 0)

    return mp_kernel(t, src2d, dst2d, ew_flat).reshape(NC, N_PAD, w)


def kernel(x, edge_index, edge_weights, W1, b1, W2, b2, W3, b3, W4, b4,
           gamma, beta, fc1_W, fc1_b, fc2_W, fc2_b, fc3_W, fc3_b):
    src = edge_index[0].astype(jnp.int32)
    dst = edge_index[1].astype(jnp.int32)
    ew = edge_weights

    pad = jnp.full((E_PAD - E,), N, jnp.int32)
    src2d = jnp.concatenate([src, pad]).reshape(E_PAD // SZ, SZ)
    dst2d = jnp.concatenate([dst, pad]).reshape(E_PAD // SZ, SZ)
    ew_flat = jnp.concatenate([ew, jnp.zeros((E_PAD - E,), jnp.float32)])

    x_pad = jnp.zeros((N_PAD, 8), jnp.float32).at[:N, :6].set(x)
    w1_pad = jnp.zeros((8, 16), jnp.float32).at[:6].set(W1)

    dego, degi = _sc_degrees(src2d, dst2d)

    t1 = _tc_t1(dego, degi, x_pad, w1_pad)
    agg1 = _sc_mp(t1, src2d, dst2d, ew_flat, 16)
    (t2,) = _tc_dense(dego, degi, [agg1], 16, None, b1, 1, 16)
    agg2 = _sc_mp(t2, src2d, dst2d, ew_flat, 16)
    (t3,) = _tc_dense(dego, degi, [agg2], 16, W2, b2, 1, 32)
    agg3 = _sc_mp(t3, src2d, dst2d, ew_flat, 32)
    t4a, t4b = _tc_dense(dego, degi, [agg3], 32, W3, b3, 2, 32)
    agg4a = _sc_mp(t4a, src2d, dst2d, ew_flat, 32)
    agg4b = _sc_mp(t4b, src2d, dst2d, ew_flat, 32)
    return _tc_head(dego, degi, [agg4a, agg4b], W4, b4, gamma, beta,
                    fc1_W, fc1_b, fc2_W, fc2_b, fc3_W, fc3_b)


# speedup vs baseline: 12.6996x; 1.0102x over previous
import functools
import jax
import jax.numpy as jnp
import numpy as np
from jax import lax
from jax.experimental import pallas as pl
from jax.experimental.pallas import tpu as pltpu
from jax.experimental.pallas import tpu_sc as plsc

N = 50000
E = 1600000
N_PAD = 50176
R = 1792
NBLK = N_PAD // R

NC = 2
NS = 16
NW = NC * NS
SZ = 128
RW = 400
E_PAD = NW * RW * SZ
K = 16
NCH = RW // K
NT = N_PAD // NS
W16 = 16


def _lrelu(v, a):
    return jnp.where(v >= 0, v, a * v)


def _scales(dego, degi):
    dout = lax.rsqrt(jnp.maximum(dego[0] + dego[1], 1.0))
    din = lax.rsqrt(jnp.maximum(degi[0] + degi[1], 1.0))
    return dout, din


def _deg_spec():
    return pl.BlockSpec((2, R), lambda i: (0, i))


def _full(shape):
    return pl.BlockSpec(shape, lambda i: tuple(0 for _ in shape))


def _t1_body(dego_ref, degi_ref, x_ref, w1_ref, o_ref):
    dout, _ = _scales(dego_ref[...], degi_ref[...])
    o_ref[...] = jnp.dot(x_ref[...] * dout[:, None], w1_ref[...],
                         preferred_element_type=jnp.float32)


def _tc_t1(dego, degi, x_pad, w1):
    return pl.pallas_call(
        _t1_body,
        grid=(NBLK,),
        in_specs=[_deg_spec(), _deg_spec(),
                  pl.BlockSpec((R, 8), lambda i: (i, 0)),
                  _full((8, 16))],
        out_specs=pl.BlockSpec((R, 16), lambda i: (i, 0)),
        out_shape=jax.ShapeDtypeStruct((N_PAD, 16), jnp.float32),
    )(dego, degi, x_pad, w1)


def _dense_body(*refs, na, aw, has_w, nout, ow):
    dego_ref, degi_ref = refs[0], refs[1]
    aggs = refs[2:2 + na]
    pos = 2 + na
    if has_w:
        w_ref = refs[pos]
        pos += 1
    b_ref = refs[pos]
    outs = refs[pos + 1:]
    dout, din = _scales(dego_ref[...], degi_ref[...])
    acc = None
    for i in range(na):
        a = aggs[i][0] + aggs[i][1]
        if has_w:
            term = jnp.dot(a, w_ref[pl.ds(i * aw, aw), :],
                           preferred_element_type=jnp.float32)
        else:
            term = a
        acc = term if acc is None else acc + term
    h = _lrelu(acc * din[:, None] + b_ref[...], 0.01) * dout[:, None]
    for c in range(nout):
        outs[c][...] = h[:, c * ow:(c + 1) * ow]


def _tc_dense(dego, degi, aggs, aw, w, b, nout, ow):
    na = len(aggs)
    has_w = w is not None
    in_specs = [_deg_spec(), _deg_spec()]
    args = [dego, degi]
    for a in aggs:
        in_specs.append(pl.BlockSpec((2, R, aw), lambda i: (0, i, 0)))
        args.append(a)
    if has_w:
        in_specs.append(_full(w.shape))
        args.append(w)
    in_specs.append(_full((1, nout * ow)))
    args.append(b.reshape(1, -1))
    return pl.pallas_call(
        functools.partial(_dense_body, na=na, aw=aw, has_w=has_w,
                          nout=nout, ow=ow),
        grid=(NBLK,),
        in_specs=in_specs,
        out_specs=[pl.BlockSpec((R, ow), lambda i: (i, 0))] * nout,
        out_shape=[jax.ShapeDtypeStruct((N_PAD, ow), jnp.float32)] * nout,
    )(*args)


def _head_body(dego_ref, degi_ref, a0_ref, a1_ref,
               w4_ref, b4_ref, gb_ref,
               f1w_ref, f1b_ref, f2w_ref, f2b_ref, f3w_ref, f3b_ref,
               o_ref, acc_ref):
    i = pl.program_id(0)
    _, din = _scales(dego_ref[...], degi_ref[...])
    aggs = (a0_ref, a1_ref)
    h = None
    for c in range(2):
        term = jnp.dot(aggs[c][0] + aggs[c][1], w4_ref[pl.ds(c * 32, 32), :],
                       preferred_element_type=jnp.float32)
        h = term if h is None else h + term
    h = h * din[:, None] + b4_ref[...]
    h = h * gb_ref[0:1, :] + gb_ref[1:2, :]
    z = _lrelu(h, 0.01)
    rows = i * R + lax.broadcasted_iota(jnp.int32, (R, 1), 0)
    z = jnp.where(rows < N, z, 0.0)
    part = jnp.sum(z, axis=0, keepdims=True)

    @pl.when(i == 0)
    def _():
        acc_ref[...] = jnp.zeros_like(acc_ref)

    acc_ref[...] += part

    @pl.when(i == NBLK - 1)
    def _():
        g = acc_ref[...]
        o = _lrelu(jnp.dot(g, f1w_ref[...]) + f1b_ref[...], 0.1)
        o = _lrelu(jnp.dot(o, f2w_ref[...]) + f2b_ref[...], 0.1)
        o_ref[...] = jnp.dot(o, f3w_ref[...]) + f3b_ref[...]


def _tc_head(dego, degi, aggs, w4, b4, gamma, beta,
             f1w, f1b, f2w, f2b, f3w, f3b):
    gb = jnp.stack([gamma * np.float32(1.0 / np.sqrt(1.0 + 1e-5)), beta])
    agg_spec = pl.BlockSpec((2, R, 32), lambda i: (0, i, 0))
    return pl.pallas_call(
        _head_body,
        grid=(NBLK,),
        in_specs=[_deg_spec(), _deg_spec(),
                  agg_spec, agg_spec,
                  _full((64, 50)), _full((1, 50)), _full((2, 50)),
                  _full((50, 30)), _full((1, 30)),
                  _full((30, 10)), _full((1, 10)),
                  _full((10, 2)), _full((1, 2))],
        out_specs=pl.BlockSpec((1, 2), lambda i: (0, 0)),
        out_shape=jax.ShapeDtypeStruct((1, 2), jnp.float32),
        scratch_shapes=[pltpu.VMEM((1, 50), jnp.float32)],
        compiler_params=pltpu.CompilerParams(
            dimension_semantics=("arbitrary",)),
    )(dego, degi, *aggs, w4, b4.reshape(1, -1), gb,
      f1w, f1b.reshape(1, -1), f2w, f2b.reshape(1, -1),
      f3w, f3b.reshape(1, -1))


_MESH = plsc.VectorSubcoreMesh(core_axis_name="c", subcore_axis_name="s")


def _sc_degrees(src2d, dst2d):
    @functools.partial(
        pl.kernel, mesh=_MESH,
        out_type=(jax.ShapeDtypeStruct((NC * N_PAD,), jnp.float32),
                  jax.ShapeDtypeStruct((NC * N_PAD,), jnp.float32)),
        compiler_params=pltpu.CompilerParams(use_tc_tiling_on_sc=False),
        scratch_types=[pltpu.VMEM((K, SZ), jnp.int32),
                       pltpu.VMEM((K, SZ), jnp.int32),
                       pltpu.VMEM((SZ,), jnp.float32),
                       pltpu.VMEM((NT,), jnp.float32),
                       pltpu.VMEM_SHARED((N_PAD,), jnp.float32),
                       pltpu.VMEM_SHARED((N_PAD,), jnp.float32),
                       pltpu.SemaphoreType.DMA,
                       pltpu.SemaphoreType.DMA,
                       pltpu.SemaphoreType.DMA,
                       pltpu.SemaphoreType.DMA])
    def deg_kernel(src_hbm, dst_hbm, dego_hbm, degi_hbm,
                   src_v, dst_v, ones_v, zer_v, dego_sh, degi_sh,
                   ds0, ds1, ds2, ds3):
        cid = lax.axis_index("c")
        sid = lax.axis_index("s")
        wid = sid * NC + cid

        def fill(i, _):
            zer_v[pl.ds(i * 16, 16)] = jnp.zeros((16,), jnp.float32)
            return 0
        lax.fori_loop(0, NT // 16, fill, 0)
        for c in range(SZ // 16):
            ones_v[pl.ds(c * 16, 16)] = jnp.full((16,), 1.0, jnp.float32)
        pltpu.sync_copy(zer_v, dego_sh.at[pl.ds(sid * NT, NT)])
        pltpu.sync_copy(zer_v, degi_sh.at[pl.ds(sid * NT, NT)])
        plsc.subcore_barrier()

        dsem = (ds0, ds1, ds2, ds3)

        def outer(g, _):
            row0 = wid * RW + g * K
            pltpu.sync_copy(src_hbm.at[pl.ds(row0, K)], src_v)
            pltpu.sync_copy(dst_hbm.at[pl.ds(row0, K)], dst_v)
            cps = []
            for i in range(2 * K):
                j = i // 2
                if i >= 4:
                    cps[i - 4].wait()
                tgt = dego_sh if i % 2 == 0 else degi_sh
                idxr = src_v if i % 2 == 0 else dst_v
                cps.append(pltpu.async_copy(ones_v, tgt.at[idxr.at[j]],
                                            dsem[i % 4], add=True))
            for cp in cps[-4:]:
                cp.wait()
            return 0
        lax.fori_loop(0, NCH, outer, 0)
        plsc.subcore_barrier()

        sl = pl.ds(sid * NT, NT)
        osl = pl.ds(cid * N_PAD + sid * NT, NT)
        pltpu.sync_copy(dego_sh.at[sl], zer_v)
        pltpu.sync_copy(zer_v, dego_hbm.at[osl])
        pltpu.sync_copy(degi_sh.at[sl], zer_v)
        pltpu.sync_copy(zer_v, degi_hbm.at[osl])

    dego, degi = deg_kernel(src2d, dst2d)
    return dego.reshape(NC, N_PAD), degi.reshape(NC, N_PAD)


def _sc_mp(t, src2d, dst2d, ew_flat, w):
    zr = 56

    @functools.partial(
        pl.kernel, mesh=_MESH,
        out_type=jax.ShapeDtypeStruct((NC * N_PAD, w), jnp.float32),
        compiler_params=pltpu.CompilerParams(use_tc_tiling_on_sc=False),
        scratch_types=[pltpu.VMEM((K, SZ), jnp.int32),
                       pltpu.VMEM((K, SZ), jnp.int32),
                       pltpu.VMEM((K, SZ), jnp.int32),
                       pltpu.VMEM((K, SZ), jnp.int32),
                       pltpu.VMEM((K * SZ,), jnp.float32),
                       pltpu.VMEM((K * SZ,), jnp.float32),
                       pltpu.VMEM((4, SZ, w), jnp.float32),
                       pltpu.VMEM((zr, w), jnp.float32),
                       pltpu.VMEM_SHARED((N_PAD, w), jnp.float32),
                       pltpu.SemaphoreType.DMA,
                       pltpu.SemaphoreType.DMA,
                       pltpu.SemaphoreType.DMA,
                       pltpu.SemaphoreType.DMA,
                       pltpu.SemaphoreType.DMA,
                       pltpu.SemaphoreType.DMA,
                       pltpu.SemaphoreType.DMA,
                       pltpu.SemaphoreType.DMA,
                       pltpu.SemaphoreType.DMA,
                       pltpu.SemaphoreType.DMA])
    def mp_kernel(t_hbm, src_hbm, dst_hbm, ew_hbm, out_hbm,
                  src_v0, src_v1, dst_v0, dst_v1, ew_v0, ew_v1,
                  rows_v, zer_v, agg_sh,
                  gs0, gs1, gs2, gs3, ss0, ss1, ss2, ss3, is0, is1):
        cid = lax.axis_index("c")
        sid = lax.axis_index("s")
        wid = sid * NC + cid
        gsem = (gs0, gs1, gs2, gs3)
        ssem = (ss0, ss1, ss2, ss3)
        isem = (is0, is1)
        srcs = (src_v0, src_v1)
        dsts = (dst_v0, dst_v1)
        ews = (ew_v0, ew_v1)

        def fill(i, _):
            for c in range(w // 16):
                zer_v[i, pl.ds(c * 16, 16)] = jnp.zeros((16,), jnp.float32)
            return 0
        lax.fori_loop(0, zr, fill, 0)

        def zero(q, _):
            pltpu.sync_copy(zer_v, agg_sh.at[pl.ds(sid * NT + q * zr, zr)])
            return 0
        lax.fori_loop(0, NT // zr, zero, 0)
        plsc.subcore_barrier()

        def fire_idx(gch, par):
            row0 = wid * RW + gch * K
            pltpu.async_copy(src_hbm.at[pl.ds(row0, K)], srcs[par],
                             isem[par])
            pltpu.async_copy(dst_hbm.at[pl.ds(row0, K)], dsts[par],
                             isem[par])
            pltpu.async_copy(ew_hbm.at[pl.ds(row0 * SZ, K * SZ)], ews[par],
                             isem[par])

        def wait_idx(par):
            pltpu.make_async_copy(src_hbm.at[pl.ds(0, K)], srcs[par],
                                  isem[par]).wait()
            pltpu.make_async_copy(dst_hbm.at[pl.ds(0, K)], dsts[par],
                                  isem[par]).wait()
            pltpu.make_async_copy(ew_hbm.at[pl.ds(0, K * SZ)], ews[par],
                                  isem[par]).wait()

        def chunk(gch, par, prefetch):
            wait_idx(par)
            if prefetch:
                fire_idx(gch + 1, 1 - par)
            src_v, dst_v, ew_v = srcs[par], dsts[par], ews[par]
            gcps = [pltpu.async_copy(t_hbm.at[src_v.at[jj]], rows_v.at[jj],
                                     gsem[jj]) for jj in range(2)]
            scps = []
            for j in range(K):
                b = j % 4
                if j >= 2:
                    scps[j - 2].wait()
                if j + 2 < K:
                    nb = (j + 2) % 4
                    gcps.append(pltpu.async_copy(
                        t_hbm.at[src_v.at[j + 2]], rows_v.at[nb], gsem[nb]))
                gcps[j].wait()

                def scale(bk, _):
                    ewv = ew_v[pl.ds(j * SZ + bk * 16, 16)]
                    for l in range(16):
                        s = ewv[l]
                        e = bk * 16 + l
                        for c in range(w // 16):
                            csl = pl.ds(c * 16, 16)
                            rows_v[b, e, csl] = rows_v[b, e, csl] * s
                    return 0
                lax.fori_loop(0, SZ // 16, scale, 0)
                scps.append(pltpu.async_copy(rows_v.at[b],
                                             agg_sh.at[dst_v.at[j]],
                                             ssem[b], add=True))
            scps[K - 2].wait()
            scps[K - 1].wait()

        fire_idx(0, 0)

        def outer(g2, _):
            chunk(2 * g2, 0, True)
            chunk(2 * g2 + 1, 1, True)
            return 0
        lax.fori_loop(0, NCH // 2, outer, 0)
        chunk(NCH - 1, 0, False)
        plsc.subcore_barrier()

        def wb(q, _):
            off = sid * NT + q * zr
            pltpu.sync_copy(agg_sh.at[pl.ds(off, zr)], zer_v)
            pltpu.sync_copy(zer_v, out_hbm.at[pl.ds(cid * N_PAD + off, zr)])
            return 0
        lax.fori_loop(0, NT // zr, wb, 0)

    return mp_kernel(t, src2d, dst2d, ew_flat).reshape(NC, N_PAD, w)


def kernel(x, edge_index, edge_weights, W1, b1, W2, b2, W3, b3, W4, b4,
           gamma, beta, fc1_W, fc1_b, fc2_W, fc2_b, fc3_W, fc3_b):
    src = edge_index[0].astype(jnp.int32)
    dst = edge_index[1].astype(jnp.int32)
    ew = edge_weights

    pad = jnp.full((E_PAD - E,), N, jnp.int32)
    src2d = jnp.concatenate([src, pad]).reshape(E_PAD // SZ, SZ)
    dst2d = jnp.concatenate([dst, pad]).reshape(E_PAD // SZ, SZ)
    ew_flat = jnp.concatenate([ew, jnp.zeros((E_PAD - E,), jnp.float32)])

    x_pad = jnp.zeros((N_PAD, 8), jnp.float32).at[:N, :6].set(x)
    w1_pad = jnp.zeros((8, 16), jnp.float32).at[:6].set(W1)

    dego, degi = _sc_degrees(src2d, dst2d)

    t1 = _tc_t1(dego, degi, x_pad, w1_pad)
    agg1 = _sc_mp(t1, src2d, dst2d, ew_flat, 16)
    (t2,) = _tc_dense(dego, degi, [agg1], 16, None, b1, 1, 16)
    agg2 = _sc_mp(t2, src2d, dst2d, ew_flat, 16)
    (t3,) = _tc_dense(dego, degi, [agg2], 16, W2, b2, 1, 32)
    agg3 = _sc_mp(t3, src2d, dst2d, ew_flat, 32)
    t4a, t4b = _tc_dense(dego, degi, [agg3], 32, W3, b3, 2, 32)
    agg4a = _sc_mp(t4a, src2d, dst2d, ew_flat, 32)
    agg4b = _sc_mp(t4b, src2d, dst2d, ew_flat, 32)
    return _tc_head(dego, degi, [agg4a, agg4b], W4, b4, gamma, beta,
                    fc1_W, fc1_b, fc2_W, fc2_b, fc3_W, fc3_b)


# speedup vs baseline: 12.7790x; 1.0063x over previous
import functools
import jax
import jax.numpy as jnp
import numpy as np
from jax import lax
from jax.experimental import pallas as pl
from jax.experimental.pallas import tpu as pltpu
from jax.experimental.pallas import tpu_sc as plsc

N = 50000
E = 1600000
N_PAD = 50176
R = 1792
NBLK = N_PAD // R

NC = 2
NS = 16
NW = NC * NS
SZ = 128
RW = 400
E_PAD = NW * RW * SZ
K = 16
NCH = RW // K
NT = N_PAD // NS
W16 = 16


def _lrelu(v, a):
    return jnp.where(v >= 0, v, a * v)


def _scales(dego, degi):
    dout = lax.rsqrt(jnp.maximum(dego[0] + dego[1], 1.0))
    din = lax.rsqrt(jnp.maximum(degi[0] + degi[1], 1.0))
    return dout, din


def _deg_spec():
    return pl.BlockSpec((2, R), lambda i: (0, i))


def _full(shape):
    return pl.BlockSpec(shape, lambda i: tuple(0 for _ in shape))


def _t1_body(dego_ref, degi_ref, x_ref, w1_ref, o_ref):
    dout, _ = _scales(dego_ref[...], degi_ref[...])
    o_ref[...] = jnp.dot(x_ref[...] * dout[:, None], w1_ref[...],
                         preferred_element_type=jnp.float32)


def _tc_t1(dego, degi, x_pad, w1):
    return pl.pallas_call(
        _t1_body,
        grid=(NBLK,),
        in_specs=[_deg_spec(), _deg_spec(),
                  pl.BlockSpec((R, 8), lambda i: (i, 0)),
                  _full((8, 16))],
        out_specs=pl.BlockSpec((R, 16), lambda i: (i, 0)),
        out_shape=jax.ShapeDtypeStruct((N_PAD, 16), jnp.float32),
    )(dego, degi, x_pad, w1)


def _dense_body(*refs, na, aw, has_w, nout, ow, out_dtype):
    dego_ref, degi_ref = refs[0], refs[1]
    aggs = refs[2:2 + na]
    pos = 2 + na
    if has_w:
        w_ref = refs[pos]
        pos += 1
    b_ref = refs[pos]
    outs = refs[pos + 1:]
    dout, din = _scales(dego_ref[...], degi_ref[...])
    acc = None
    for i in range(na):
        a = aggs[i][0] + aggs[i][1]
        if has_w:
            term = jnp.dot(a, w_ref[pl.ds(i * aw, aw), :],
                           preferred_element_type=jnp.float32)
        else:
            term = a
        acc = term if acc is None else acc + term
    h = _lrelu(acc * din[:, None] + b_ref[...], 0.01) * dout[:, None]
    for c in range(nout):
        outs[c][...] = h[:, c * ow:(c + 1) * ow].astype(out_dtype)


def _tc_dense(dego, degi, aggs, aw, w, b, nout, ow,
              out_dtype=jnp.float32):
    na = len(aggs)
    has_w = w is not None
    in_specs = [_deg_spec(), _deg_spec()]
    args = [dego, degi]
    for a in aggs:
        in_specs.append(pl.BlockSpec((2, R, aw), lambda i: (0, i, 0)))
        args.append(a)
    if has_w:
        in_specs.append(_full(w.shape))
        args.append(w)
    in_specs.append(_full((1, nout * ow)))
    args.append(b.reshape(1, -1))
    return pl.pallas_call(
        functools.partial(_dense_body, na=na, aw=aw, has_w=has_w,
                          nout=nout, ow=ow, out_dtype=out_dtype),
        grid=(NBLK,),
        in_specs=in_specs,
        out_specs=[pl.BlockSpec((R, ow), lambda i: (i, 0))] * nout,
        out_shape=[jax.ShapeDtypeStruct((N_PAD, ow), out_dtype)] * nout,
    )(*args)


def _head_body(dego_ref, degi_ref, a0_ref, a1_ref,
               w4_ref, b4_ref, gb_ref,
               f1w_ref, f1b_ref, f2w_ref, f2b_ref, f3w_ref, f3b_ref,
               o_ref, acc_ref):
    i = pl.program_id(0)
    _, din = _scales(dego_ref[...], degi_ref[...])
    aggs = (a0_ref, a1_ref)
    h = None
    for c in range(2):
        term = jnp.dot(aggs[c][0] + aggs[c][1], w4_ref[pl.ds(c * 32, 32), :],
                       preferred_element_type=jnp.float32)
        h = term if h is None else h + term
    h = h * din[:, None] + b4_ref[...]
    h = h * gb_ref[0:1, :] + gb_ref[1:2, :]
    z = _lrelu(h, 0.01)
    rows = i * R + lax.broadcasted_iota(jnp.int32, (R, 1), 0)
    z = jnp.where(rows < N, z, 0.0)
    part = jnp.sum(z, axis=0, keepdims=True)

    @pl.when(i == 0)
    def _():
        acc_ref[...] = jnp.zeros_like(acc_ref)

    acc_ref[...] += part

    @pl.when(i == NBLK - 1)
    def _():
        g = acc_ref[...]
        o = _lrelu(jnp.dot(g, f1w_ref[...]) + f1b_ref[...], 0.1)
        o = _lrelu(jnp.dot(o, f2w_ref[...]) + f2b_ref[...], 0.1)
        o_ref[...] = jnp.dot(o, f3w_ref[...]) + f3b_ref[...]


def _tc_head(dego, degi, aggs, w4, b4, gamma, beta,
             f1w, f1b, f2w, f2b, f3w, f3b):
    gb = jnp.stack([gamma * np.float32(1.0 / np.sqrt(1.0 + 1e-5)), beta])
    agg_spec = pl.BlockSpec((2, R, 32), lambda i: (0, i, 0))
    return pl.pallas_call(
        _head_body,
        grid=(NBLK,),
        in_specs=[_deg_spec(), _deg_spec(),
                  agg_spec, agg_spec,
                  _full((64, 50)), _full((1, 50)), _full((2, 50)),
                  _full((50, 30)), _full((1, 30)),
                  _full((30, 10)), _full((1, 10)),
                  _full((10, 2)), _full((1, 2))],
        out_specs=pl.BlockSpec((1, 2), lambda i: (0, 0)),
        out_shape=jax.ShapeDtypeStruct((1, 2), jnp.float32),
        scratch_shapes=[pltpu.VMEM((1, 50), jnp.float32)],
        compiler_params=pltpu.CompilerParams(
            dimension_semantics=("arbitrary",)),
    )(dego, degi, *aggs, w4, b4.reshape(1, -1), gb,
      f1w, f1b.reshape(1, -1), f2w, f2b.reshape(1, -1),
      f3w, f3b.reshape(1, -1))


_MESH = plsc.VectorSubcoreMesh(core_axis_name="c", subcore_axis_name="s")


def _sc_degrees(src2d, dst2d):
    @functools.partial(
        pl.kernel, mesh=_MESH,
        out_type=(jax.ShapeDtypeStruct((NC * N_PAD,), jnp.float32),
                  jax.ShapeDtypeStruct((NC * N_PAD,), jnp.float32)),
        compiler_params=pltpu.CompilerParams(use_tc_tiling_on_sc=False),
        scratch_types=[pltpu.VMEM((K, SZ), jnp.int32),
                       pltpu.VMEM((K, SZ), jnp.int32),
                       pltpu.VMEM((SZ,), jnp.float32),
                       pltpu.VMEM((NT,), jnp.float32),
                       pltpu.VMEM_SHARED((N_PAD,), jnp.float32),
                       pltpu.VMEM_SHARED((N_PAD,), jnp.float32),
                       pltpu.SemaphoreType.DMA,
                       pltpu.SemaphoreType.DMA,
                       pltpu.SemaphoreType.DMA,
                       pltpu.SemaphoreType.DMA])
    def deg_kernel(src_hbm, dst_hbm, dego_hbm, degi_hbm,
                   src_v, dst_v, ones_v, zer_v, dego_sh, degi_sh,
                   ds0, ds1, ds2, ds3):
        cid = lax.axis_index("c")
        sid = lax.axis_index("s")
        wid = sid * NC + cid

        def fill(i, _):
            zer_v[pl.ds(i * 16, 16)] = jnp.zeros((16,), jnp.float32)
            return 0
        lax.fori_loop(0, NT // 16, fill, 0)
        for c in range(SZ // 16):
            ones_v[pl.ds(c * 16, 16)] = jnp.full((16,), 1.0, jnp.float32)
        pltpu.sync_copy(zer_v, dego_sh.at[pl.ds(sid * NT, NT)])
        pltpu.sync_copy(zer_v, degi_sh.at[pl.ds(sid * NT, NT)])
        plsc.subcore_barrier()

        dsem = (ds0, ds1, ds2, ds3)

        def outer(g, _):
            row0 = wid * RW + g * K
            pltpu.sync_copy(src_hbm.at[pl.ds(row0, K)], src_v)
            pltpu.sync_copy(dst_hbm.at[pl.ds(row0, K)], dst_v)
            cps = []
            for i in range(2 * K):
                j = i // 2
                if i >= 4:
                    cps[i - 4].wait()
                tgt = dego_sh if i % 2 == 0 else degi_sh
                idxr = src_v if i % 2 == 0 else dst_v
                cps.append(pltpu.async_copy(ones_v, tgt.at[idxr.at[j]],
                                            dsem[i % 4], add=True))
            for cp in cps[-4:]:
                cp.wait()
            return 0
        lax.fori_loop(0, NCH, outer, 0)
        plsc.subcore_barrier()

        sl = pl.ds(sid * NT, NT)
        osl = pl.ds(cid * N_PAD + sid * NT, NT)
        pltpu.sync_copy(dego_sh.at[sl], zer_v)
        pltpu.sync_copy(zer_v, dego_hbm.at[osl])
        pltpu.sync_copy(degi_sh.at[sl], zer_v)
        pltpu.sync_copy(zer_v, degi_hbm.at[osl])

    dego, degi = deg_kernel(src2d, dst2d)
    return dego.reshape(NC, N_PAD), degi.reshape(NC, N_PAD)


def _sc_mp(t, src2d, dst2d, ew_flat, w):
    zr = 56
    rd = 6 if w == 16 else 4
    pd = rd - 2

    @functools.partial(
        pl.kernel, mesh=_MESH,
        out_type=jax.ShapeDtypeStruct((NC * N_PAD, w), jnp.float32),
        compiler_params=pltpu.CompilerParams(use_tc_tiling_on_sc=False),
        scratch_types=[pltpu.VMEM((K, SZ), jnp.int32),
                       pltpu.VMEM((K, SZ), jnp.int32),
                       pltpu.VMEM((K, SZ), jnp.int32),
                       pltpu.VMEM((K, SZ), jnp.int32),
                       pltpu.VMEM((K * SZ,), jnp.float32),
                       pltpu.VMEM((K * SZ,), jnp.float32),
                       pltpu.VMEM((rd, SZ, w), jnp.float32),
                       pltpu.VMEM((zr, w), jnp.float32),
                       pltpu.VMEM_SHARED((N_PAD, w), jnp.float32)]
                      + [pltpu.SemaphoreType.DMA] * (2 * rd + 2))
    def mp_kernel(t_hbm, src_hbm, dst_hbm, ew_hbm, out_hbm,
                  src_v0, src_v1, dst_v0, dst_v1, ew_v0, ew_v1,
                  rows_v, zer_v, agg_sh, *sems):
        cid = lax.axis_index("c")
        sid = lax.axis_index("s")
        wid = sid * NC + cid
        gsem = sems[:rd]
        ssem = sems[rd:2 * rd]
        isem = sems[2 * rd:]
        srcs = (src_v0, src_v1)
        dsts = (dst_v0, dst_v1)
        ews = (ew_v0, ew_v1)

        def fill(i, _):
            for c in range(w // 16):
                zer_v[i, pl.ds(c * 16, 16)] = jnp.zeros((16,), jnp.float32)
            return 0
        lax.fori_loop(0, zr, fill, 0)

        def zero(q, _):
            pltpu.sync_copy(zer_v, agg_sh.at[pl.ds(sid * NT + q * zr, zr)])
            return 0
        lax.fori_loop(0, NT // zr, zero, 0)
        plsc.subcore_barrier()

        def fire_idx(gch, par):
            row0 = wid * RW + gch * K
            pltpu.async_copy(src_hbm.at[pl.ds(row0, K)], srcs[par],
                             isem[par])
            pltpu.async_copy(dst_hbm.at[pl.ds(row0, K)], dsts[par],
                             isem[par])
            pltpu.async_copy(ew_hbm.at[pl.ds(row0 * SZ, K * SZ)], ews[par],
                             isem[par])

        def wait_idx(par):
            pltpu.make_async_copy(src_hbm.at[pl.ds(0, K)], srcs[par],
                                  isem[par]).wait()
            pltpu.make_async_copy(dst_hbm.at[pl.ds(0, K)], dsts[par],
                                  isem[par]).wait()
            pltpu.make_async_copy(ew_hbm.at[pl.ds(0, K * SZ)], ews[par],
                                  isem[par]).wait()

        def chunk(gch, par, prefetch):
            wait_idx(par)
            if prefetch:
                fire_idx(gch + 1, 1 - par)
            src_v, dst_v, ew_v = srcs[par], dsts[par], ews[par]
            gcps = [pltpu.async_copy(t_hbm.at[src_v.at[jj]], rows_v.at[jj],
                                     gsem[jj]) for jj in range(pd)]
            scps = []
            for j in range(K):
                b = j % rd
                if j >= 2:
                    scps[j - 2].wait()
                if j + pd < K:
                    nb = (j + pd) % rd
                    gcps.append(pltpu.async_copy(
                        t_hbm.at[src_v.at[j + pd]], rows_v.at[nb], gsem[nb]))
                gcps[j].wait()

                def scale(bk, _):
                    ewv = ew_v[pl.ds(j * SZ + bk * 16, 16)]
                    for l in range(16):
                        s = ewv[l]
                        e = bk * 16 + l
                        for c in range(w // 16):
                            csl = pl.ds(c * 16, 16)
                            rows_v[b, e, csl] = rows_v[b, e, csl] * s
                    return 0
                lax.fori_loop(0, SZ // 16, scale, 0)
                scps.append(pltpu.async_copy(rows_v.at[b],
                                             agg_sh.at[dst_v.at[j]],
                                             ssem[b], add=True))
            scps[K - 2].wait()
            scps[K - 1].wait()

        fire_idx(0, 0)

        def outer(g2, _):
            chunk(2 * g2, 0, True)
            chunk(2 * g2 + 1, 1, True)
            return 0
        lax.fori_loop(0, NCH // 2, outer, 0)
        chunk(NCH - 1, 0, False)
        plsc.subcore_barrier()

        def wb(q, _):
            off = sid * NT + q * zr
            pltpu.sync_copy(agg_sh.at[pl.ds(off, zr)], zer_v)
            pltpu.sync_copy(zer_v, out_hbm.at[pl.ds(cid * N_PAD + off, zr)])
            return 0
        lax.fori_loop(0, NT // zr, wb, 0)

    return mp_kernel(t, src2d, dst2d, ew_flat).reshape(NC, N_PAD, w)


def kernel(x, edge_index, edge_weights, W1, b1, W2, b2, W3, b3, W4, b4,
           gamma, beta, fc1_W, fc1_b, fc2_W, fc2_b, fc3_W, fc3_b):
    src = edge_index[0].astype(jnp.int32)
    dst = edge_index[1].astype(jnp.int32)
    ew = edge_weights

    pad = jnp.full((E_PAD - E,), N, jnp.int32)
    src2d = jnp.concatenate([src, pad]).reshape(E_PAD // SZ, SZ)
    dst2d = jnp.concatenate([dst, pad]).reshape(E_PAD // SZ, SZ)
    ew_flat = jnp.concatenate([ew, jnp.zeros((E_PAD - E,), jnp.float32)])

    x_pad = jnp.zeros((N_PAD, 8), jnp.float32).at[:N, :6].set(x)
    w1_pad = jnp.zeros((8, 16), jnp.float32).at[:6].set(W1)

    dego, degi = _sc_degrees(src2d, dst2d)

    t1 = _tc_t1(dego, degi, x_pad, w1_pad)
    agg1 = _sc_mp(t1, src2d, dst2d, ew_flat, 16)
    (t2,) = _tc_dense(dego, degi, [agg1], 16, None, b1, 1, 16)
    agg2 = _sc_mp(t2, src2d, dst2d, ew_flat, 16)
    (t3,) = _tc_dense(dego, degi, [agg2], 16, W2, b2, 1, 32)
    agg3 = _sc_mp(t3, src2d, dst2d, ew_flat, 32)
    t4a, t4b = _tc_dense(dego, degi, [agg3], 32, W3, b3, 2, 32)
    agg4a = _sc_mp(t4a, src2d, dst2d, ew_flat, 32)
    agg4b = _sc_mp(t4b, src2d, dst2d, ew_flat, 32)
    return _tc_head(dego, degi, [agg4a, agg4b], W4, b4, gamma, beta,
                    fc1_W, fc1_b, fc2_W, fc2_b, fc3_W, fc3_b)
